# baseline (speedup 1.0000x reference)
import functools

import jax
import jax.numpy as jnp
from jax import lax
from jax.experimental import pallas as pl
from jax.experimental.pallas import tpu as pltpu
from jax.experimental.pallas import tpu_sc as plsc

N = 100128
E = 1602048
D = 32
NW = 32
NPAD = 102400
NR_W = NPAD // NW
EPAD = 1605632
ER_W = EPAD // NW
ER_T = EPAD // 16
HALF = NPAD // 2
QR = NPAD // 4
SEGW = 464

_mesh_cache = []


def _MESH():
    if not _mesh_cache:
        _mesh_cache.append(plsc.VectorSubcoreMesh(
            core_axis_name="c", subcore_axis_name="s",
            num_cores=2, num_subcores=16))
    return _mesh_cache[0]


def _wid():
    return lax.axis_index("s") * 2 + lax.axis_index("c")


_SC_PARAMS = dict(use_tc_tiling_on_sc=False)


def _emb_gather(atoms_f, embed):
    @functools.partial(
        pl.kernel,
        out_type=jax.ShapeDtypeStruct((NPAD, D), jnp.float32),
        mesh=_MESH(),
        compiler_params=pltpu.CompilerParams(**_SC_PARAMS),
        scratch_types=[
            pltpu.VMEM((NR_W,), jnp.int32),
            pltpu.VMEM((NR_W, D), jnp.float32),
            pltpu.SemaphoreType.DMA,
        ],
    )
    def k(atoms_hbm, embed_hbm, out_hbm, idx_v, rows_v, sem):
        w = _wid()
        pltpu.sync_copy(atoms_hbm.at[pl.ds(w * NR_W, NR_W)], idx_v)
        pltpu.async_copy(embed_hbm.at[idx_v], rows_v, sem).wait()
        pltpu.sync_copy(rows_v, out_hbm.at[pl.ds(w * NR_W, NR_W)])

    return k(atoms_f, embed)


def _gather2(h, src_f, dst_f):
    CR = 896
    NCH = ER_W // CR

    @functools.partial(
        pl.kernel,
        out_type=[jax.ShapeDtypeStruct((EPAD, D), jnp.float32),
                  jax.ShapeDtypeStruct((EPAD, D), jnp.float32)],
        mesh=_MESH(),
        compiler_params=pltpu.CompilerParams(**_SC_PARAMS),
        scratch_types=[
            pltpu.VMEM((CR,), jnp.int32), pltpu.VMEM((CR,), jnp.int32),
            pltpu.VMEM((CR,), jnp.int32), pltpu.VMEM((CR,), jnp.int32),
            pltpu.VMEM((CR, D), jnp.float32), pltpu.VMEM((CR, D), jnp.float32),
            pltpu.VMEM((CR, D), jnp.float32), pltpu.VMEM((CR, D), jnp.float32),
            pltpu.SemaphoreType.DMA,
            pltpu.SemaphoreType.DMA,
        ],
    )
    def k(h_hbm, src_hbm, dst_hbm, gs_hbm, gd_hbm,
          ixs0, ixs1, ixd0, ixd1, rs0, rs1, rd0, rd1, gsem, wsem):
        w = _wid()
        bufs = [(ixs0, ixd0, rs0, rd0), (ixs1, ixd1, rs1, rd1)]

        def chunk(ci, p, drain):
            ixs, ixd, rs, rd = bufs[p]
            e0 = w * ER_W + ci * CR
            pltpu.sync_copy(src_hbm.at[pl.ds(e0, CR)], ixs)
            pltpu.sync_copy(dst_hbm.at[pl.ds(e0, CR)], ixd)
            if drain:
                pltpu.make_async_copy(gs_hbm.at[pl.ds(0, CR)], rs, wsem).wait()
                pltpu.make_async_copy(gs_hbm.at[pl.ds(0, CR)], rd, wsem).wait()
            g1 = pltpu.async_copy(h_hbm.at[ixs], rs, gsem)
            g2 = pltpu.async_copy(h_hbm.at[ixd], rd, gsem)
            g1.wait()
            g2.wait()
            pltpu.async_copy(rs, gs_hbm.at[pl.ds(e0, CR)], wsem)
            pltpu.async_copy(rd, gd_hbm.at[pl.ds(e0, CR)], wsem)

        chunk(0, 0, False)
        chunk(1, 1, False)

        @pl.loop(2, NCH, step=2)
        def _loop(base):
            chunk(base, 0, True)
            chunk(base + 1, 1, True)

        for p in range(2):
            _, _, rs, rd = bufs[p]
            pltpu.make_async_copy(gs_hbm.at[pl.ds(0, CR)], rs, wsem).wait()
            pltpu.make_async_copy(gs_hbm.at[pl.ds(0, CR)], rd, wsem).wait()

    return k(h, src_f, dst_f)


def _scatter_part(act, src_f, kbase):
    CR = 1024
    NCH = ER_T // CR
    ACC = QR + 256
    TR = ACC // 16
    WB = 800
    CPAD = CR + 160

    @functools.partial(
        pl.kernel,
        out_type=jax.ShapeDtypeStruct((2 * QR, D), jnp.float32),
        mesh=_MESH(),
        compiler_params=pltpu.CompilerParams(
            needs_layout_passes=False, **_SC_PARAMS),
        scratch_types=[
            pltpu.VMEM((CR,), jnp.int32),
            pltpu.VMEM((CPAD,), jnp.int32), pltpu.VMEM((CPAD,), jnp.int32),
            pltpu.VMEM((CPAD,), jnp.int32), pltpu.VMEM((CPAD,), jnp.int32),
            pltpu.VMEM((CR, D), jnp.float32), pltpu.VMEM((CR, D), jnp.float32),
            pltpu.VMEM_SHARED((ACC, D), jnp.float32),
            pltpu.SemaphoreType.DMA,
            pltpu.SemaphoreType.DMA,
        ],
    )
    def k(act_hbm, src_hbm, out_hbm, ix, ce0, ce1, cl0, cl1, v0, v1,
          acc_sh, gsem, asem):
        c = lax.axis_index("c")
        s = lax.axis_index("s")
        base = kbase + c * QR
        zero16 = jnp.zeros((16,), jnp.float32)
        zero16i = jnp.zeros((16,), jnp.int32)
        dump16 = jnp.full((16,), QR, jnp.int32)
        iota16 = lax.iota(jnp.int32, 16)
        bufs = [(ce0, cl0, v0), (ce1, cl1, v1)]

        def zrow(r, carry):
            v0[r, pl.ds(0, 16)] = zero16
            v0[r, pl.ds(16, 16)] = zero16
            return carry

        lax.fori_loop(0, CR, zrow, 0)
        pltpu.sync_copy(v0, acc_sh.at[pl.ds(s * TR, CR)])
        pltpu.sync_copy(v0.at[pl.ds(0, TR - CR)],
                        acc_sh.at[pl.ds(s * TR + CR, TR - CR)])
        plsc.subcore_barrier()

        def drain_adds(cn, p):
            vv = bufs[p][2]
            for j in range(8):
                @pl.when(j * 128 < cn)
                def _():
                    pltpu.make_async_copy(
                        act_hbm.at[pl.ds(0, 128)],
                        vv.at[pl.ds(0, 128)], asem).wait()

        def compact(ci, p):
            ce, cl, vv = bufs[p]
            e0 = s * ER_T + ci * CR
            pltpu.sync_copy(src_hbm.at[pl.ds(e0, CR)], ix)

            def sc16(t, cnt):
                v = ix[pl.ds(t * 16, 16)]
                ok = (v >= base) & (v < base + QR)
                plsc.store_compressed(ce.at[pl.ds(cnt, 16)],
                                      iota16 + (e0 + t * 16), mask=ok)
                plsc.store_compressed(cl.at[pl.ds(cnt, 16)], v - base,
                                      mask=ok)
                return cnt + jnp.sum(ok.astype(jnp.int32))

            cnt = lax.fori_loop(0, CR // 16, sc16, 0)
            for j in range(8):
                ce[pl.ds(cnt + j * 16, 16)] = zero16i
                cl[pl.ds(cnt + j * 16, 16)] = dump16
            for j in range(8):
                @pl.when(j * 128 < cnt)
                def _():
                    pltpu.async_copy(act_hbm.at[ce.at[pl.ds(j * 128, 128)]],
                                     vv.at[pl.ds(j * 128, 128)], gsem)
            return cnt

        def finish(cn, p):
            ce, cl, vv = bufs[p]
            for j in range(8):
                @pl.when(j * 128 < cn)
                def _():
                    pltpu.make_async_copy(
                        act_hbm.at[pl.ds(0, 128)],
                        vv.at[pl.ds(0, 128)], gsem).wait()
            for j in range(8):
                @pl.when(j * 128 < cn)
                def _():
                    pltpu.async_copy(vv.at[pl.ds(j * 128, 128)],
                                     acc_sh.at[cl.at[pl.ds(j * 128, 128)]],
                                     asem, add=True)

        @pl.loop(0, NCH, step=2, init_carry=(0, 0))
        def _loop(b2, carry):
            n1, n2 = carry
            drain_adds(n2, 0)
            c0 = compact(b2, 0)
            finish(n1, 1)
            drain_adds(n1, 1)
            c1 = compact(b2 + 1, 1)
            finish(c0, 0)
            return c1, c0

        n1, n2 = _loop
        finish(n1, 1)
        drain_adds(n2, 0)
        drain_adds(n1, 1)
        plsc.subcore_barrier()

        for t in range(2):
            pltpu.sync_copy(acc_sh.at[pl.ds(s * (2 * WB) + t * WB, WB)],
                            v0.at[pl.ds(0, WB)])
            pltpu.sync_copy(v0.at[pl.ds(0, WB)],
                            out_hbm.at[pl.ds(c * QR + s * (2 * WB) + t * WB, WB)])

    return k(act, src_f)


def _scatter_add(act, src_f):
    lo = _scatter_part(act, src_f, 0)
    hi = _scatter_part(act, src_f, HALF)
    return jnp.concatenate([lo, hi], axis=0)


def _degree_part(src_f, kbase):
    CR = 1024
    NCH = ER_T // CR
    ACC = QR + 256
    TR = ACC // 16
    WB = 800
    CPAD = CR + 160

    @functools.partial(
        pl.kernel,
        out_type=jax.ShapeDtypeStruct((2 * QR, D), jnp.float32),
        mesh=_MESH(),
        compiler_params=pltpu.CompilerParams(
            needs_layout_passes=False, **_SC_PARAMS),
        scratch_types=[
            pltpu.VMEM((CR,), jnp.int32),
            pltpu.VMEM((CPAD,), jnp.int32), pltpu.VMEM((CPAD,), jnp.int32),
            pltpu.VMEM((CR, D), jnp.float32),
            pltpu.VMEM_SHARED((ACC, D), jnp.float32),
            pltpu.SemaphoreType.DMA,
        ],
    )
    def k(src_hbm, out_hbm, ix, cl0, cl1, ones_v, acc_sh, asem):
        c = lax.axis_index("c")
        s = lax.axis_index("s")
        base = kbase + c * QR
        zero16 = jnp.zeros((16,), jnp.float32)
        one16 = jnp.ones((16,), jnp.float32)
        dump16 = jnp.full((16,), QR, jnp.int32)
        clbufs = [cl0, cl1]

        def zrow(r, carry):
            ones_v[r, pl.ds(0, 16)] = zero16
            ones_v[r, pl.ds(16, 16)] = zero16
            return carry

        lax.fori_loop(0, CR, zrow, 0)
        pltpu.sync_copy(ones_v, acc_sh.at[pl.ds(s * TR, CR)])
        pltpu.sync_copy(ones_v.at[pl.ds(0, TR - CR)],
                        acc_sh.at[pl.ds(s * TR + CR, TR - CR)])

        def orow(r, carry):
            ones_v[r, pl.ds(0, 16)] = one16
            ones_v[r, pl.ds(16, 16)] = one16
            return carry

        lax.fori_loop(0, CR, orow, 0)
        plsc.subcore_barrier()

        def chunk(ci, p, prev):
            cl = clbufs[p]
            e0 = s * ER_T + ci * CR
            for j in range(8):
                @pl.when(j * 128 < prev)
                def _():
                    pltpu.make_async_copy(
                        out_hbm.at[pl.ds(0, 128)],
                        ones_v.at[pl.ds(0, 128)], asem).wait()
            pltpu.sync_copy(src_hbm.at[pl.ds(e0, CR)], ix)

            def sc16(t, cnt):
                v = ix[pl.ds(t * 16, 16)]
                ok = (v >= base) & (v < base + QR)
                plsc.store_compressed(cl.at[pl.ds(cnt, 16)], v - base, mask=ok)
                return cnt + jnp.sum(ok.astype(jnp.int32))

            cnt = lax.fori_loop(0, CR // 16, sc16, 0)
            for j in range(8):
                cl[pl.ds(cnt + j * 16, 16)] = dump16
            for j in range(8):
                @pl.when(j * 128 < cnt)
                def _():
                    pltpu.async_copy(ones_v.at[pl.ds(0, 128)],
                                     acc_sh.at[cl.at[pl.ds(j * 128, 128)]],
                                     asem, add=True)
            return cnt

        @pl.loop(0, NCH, step=2, init_carry=(0, 0))
        def _loop(b2, carry):
            p0, p1 = carry
            n0 = chunk(b2, 0, p0)
            n1 = chunk(b2 + 1, 1, p1)
            return n0, n1

        fin = _loop
        for p in range(2):
            prev = fin[p]
            for j in range(8):
                @pl.when(j * 128 < prev)
                def _():
                    pltpu.make_async_copy(
                        out_hbm.at[pl.ds(0, 128)],
                        ones_v.at[pl.ds(0, 128)], asem).wait()
        plsc.subcore_barrier()

        for t in range(2):
            pltpu.sync_copy(acc_sh.at[pl.ds(s * (2 * WB) + t * WB, WB)],
                            ones_v.at[pl.ds(0, WB)])
            pltpu.sync_copy(ones_v.at[pl.ds(0, WB)],
                            out_hbm.at[pl.ds(c * QR + s * (2 * WB) + t * WB, WB)])

    return k(src_f)


def _degree(src_f):
    lo = _degree_part(src_f, 0)
    hi = _degree_part(src_f, HALF)
    return jnp.concatenate([lo, hi], axis=0)


def _segment_sum(y_f, seg_f):
    PER_LANE = NR_W // 16

    @functools.partial(
        pl.kernel,
        out_type=jax.ShapeDtypeStruct((NW * SEGW,), jnp.float32),
        mesh=_MESH(),
        compiler_params=pltpu.CompilerParams(
            needs_layout_passes=False, **_SC_PARAMS),
        scratch_types=[
            pltpu.VMEM((NR_W,), jnp.float32),
            pltpu.VMEM((NR_W,), jnp.int32),
            pltpu.VMEM((16, SEGW), jnp.float32),
            pltpu.VMEM((SEGW,), jnp.float32),
        ],
    )
    def k(y_hbm, seg_hbm, out_hbm, y_v, seg_v, acc_v, pout_v):
        w = _wid()
        pltpu.sync_copy(y_hbm.at[pl.ds(w * NR_W, NR_W)], y_v)
        pltpu.sync_copy(seg_hbm.at[pl.ds(w * NR_W, NR_W)], seg_v)

        zero16 = jnp.zeros((16,), jnp.float32)

        def zacc(kk, carry):
            for r in range(16):
                acc_v[r, pl.ds(kk * 16, 16)] = zero16
            return carry

        lax.fori_loop(0, SEGW // 16, zacc, 0)

        lane = lax.iota(jnp.int32, 16)
        gbase = lane * PER_LANE

        def step(j, carry):
            cur, acc = carry
            g = gbase + j
            yv = plsc.load_gather(y_v, [g])
            sg = plsc.load_gather(seg_v, [g])
            changed = sg != cur
            plsc.addupdate_scatter(acc_v, [lane, cur], acc, mask=changed)
            acc = jnp.where(changed, yv, acc + yv)
            return sg, acc

        cur0 = jnp.full((16,), 460, jnp.int32)
        cur, acc = lax.fori_loop(0, PER_LANE, step, (cur0, zero16))
        plsc.addupdate_scatter(acc_v, [lane, cur], acc)

        def red(kk, carry):
            t = acc_v[0, pl.ds(kk * 16, 16)]
            for r in range(1, 16):
                t = t + acc_v[r, pl.ds(kk * 16, 16)]
            pout_v[pl.ds(kk * 16, 16)] = t
            return carry

        lax.fori_loop(0, SEGW // 16, red, 0)
        pltpu.sync_copy(pout_v, out_hbm.at[pl.ds(w * SEGW, SEGW)])

    return k(y_f, seg_f)


def _block_diag(w, copies):
    ki, ko = w.shape
    out = jnp.zeros((copies * ki, copies * ko), w.dtype)
    for i in range(copies):
        out = out.at[i * ki:(i + 1) * ki, i * ko:(i + 1) * ko].set(w)
    return out


def _edge_mlp(gs4, gd4, W1b, b1b):
    BLK = 2048
    R = EPAD // 4
    grid = (R // BLK,)

    def body(xs_ref, xd_ref, w_ref, b_ref, o_ref):
        x = xs_ref[...] * xd_ref[...]
        y = jnp.dot(x, w_ref[...], preferred_element_type=jnp.float32)
        o_ref[...] = jnp.maximum(y + b_ref[...], 0.0)

    return pl.pallas_call(
        body,
        grid=grid,
        in_specs=[
            pl.BlockSpec((BLK, 128), lambda i: (i, 0)),
            pl.BlockSpec((BLK, 128), lambda i: (i, 0)),
            pl.BlockSpec((128, 128), lambda i: (0, 0)),
            pl.BlockSpec((1, 128), lambda i: (0, 0)),
        ],
        out_specs=pl.BlockSpec((BLK, 128), lambda i: (i, 0)),
        out_shape=jax.ShapeDtypeStruct((R, 128), jnp.float32),
    )(gs4, gd4, W1b, b1b)


def _node_update(h4, S4, deg4, W2b, b2b, Wu1b, bu1b, Wu2b, bu2b):
    BLK = 1024
    R = NPAD // 4
    grid = (R // BLK,)

    def body(h_ref, s_ref, d_ref, w2_ref, b2_ref, wu1_ref, bu1_ref,
             wu2_ref, bu2_ref, o_ref):
        nm = jnp.dot(s_ref[...], w2_ref[...],
                     preferred_element_type=jnp.float32)
        nm = nm + d_ref[...] * b2_ref[...]
        t = jnp.maximum(
            jnp.dot(nm, wu1_ref[...], preferred_element_type=jnp.float32)
            + bu1_ref[...], 0.0)
        o_ref[...] = (h_ref[...]
                      + jnp.dot(t, wu2_ref[...],
                                preferred_element_type=jnp.float32)
                      + bu2_ref[...])

    wspec = pl.BlockSpec((128, 128), lambda i: (0, 0))
    bspec = pl.BlockSpec((1, 128), lambda i: (0, 0))
    return pl.pallas_call(
        body,
        grid=grid,
        in_specs=[
            pl.BlockSpec((BLK, 128), lambda i: (i, 0)),
            pl.BlockSpec((BLK, 128), lambda i: (i, 0)),
            pl.BlockSpec((BLK, 128), lambda i: (i, 0)),
            wspec, bspec, wspec, bspec, wspec, bspec,
        ],
        out_specs=pl.BlockSpec((BLK, 128), lambda i: (i, 0)),
        out_shape=jax.ShapeDtypeStruct((R, 128), jnp.float32),
    )(h4, S4, deg4, W2b, b2b, Wu1b, bu1b, Wu2b, bu2b)


def _readout(h4, oth4, A, Bm, br1q, C, br2q):
    BLK = 1024
    R = NPAD // 4
    grid = (R // BLK,)

    def body(h_ref, o_ref, a_ref, b_ref, br1_ref, c_ref, br2_ref, out_ref):
        y = (jnp.dot(h_ref[...], a_ref[...], preferred_element_type=jnp.float32)
             + jnp.dot(o_ref[...], b_ref[...], preferred_element_type=jnp.float32)
             + br1_ref[...])
        y = jnp.maximum(y, 0.0)
        out_ref[...] = (jnp.dot(y, c_ref[...], preferred_element_type=jnp.float32)
                        + br2_ref[...])

    return pl.pallas_call(
        body,
        grid=grid,
        in_specs=[
            pl.BlockSpec((BLK, 128), lambda i: (i, 0)),
            pl.BlockSpec((BLK, 64), lambda i: (i, 0)),
            pl.BlockSpec((128, 192), lambda i: (0, 0)),
            pl.BlockSpec((64, 192), lambda i: (0, 0)),
            pl.BlockSpec((1, 192), lambda i: (0, 0)),
            pl.BlockSpec((192, 4), lambda i: (0, 0)),
            pl.BlockSpec((1, 4), lambda i: (0, 0)),
        ],
        out_specs=pl.BlockSpec((BLK, 4), lambda i: (i, 0)),
        out_shape=jax.ShapeDtypeStruct((R, 4), jnp.float32),
    )(h4, oth4, A, Bm, br1q, C, br2q)


def kernel(encoded_atoms, edges, natoms, other_features, embed,
           Wm1, bm1, Wm2, bm2, Wu1, bu1, Wu2, bu2, Wr1, br1, Wr2, br2):
    f32 = jnp.float32
    atoms_f = jnp.pad(encoded_atoms.astype(jnp.int32), (0, NPAD - N))
    src_f = jnp.pad(edges[0].astype(jnp.int32), (0, EPAD - E),
                    constant_values=N)
    dst_f = jnp.pad(edges[1].astype(jnp.int32), (0, EPAD - E),
                    constant_values=N)

    nb = natoms.shape[0]
    seg = jnp.repeat(jnp.arange(nb, dtype=jnp.int32), natoms,
                     total_repeat_length=N)
    seg_f = jnp.pad(seg, (0, NPAD - N), constant_values=450)

    oth = jnp.pad(other_features.astype(f32), ((0, NPAD - N), (0, 0)))
    oth4 = oth.reshape(NPAD // 4, 64)

    W1b = _block_diag(Wm1.astype(f32), 4)
    b1b = jnp.tile(bm1.astype(f32), 4).reshape(1, 128)
    W2b = _block_diag(Wm2.astype(f32), 4)
    b2b = jnp.tile(bm2.astype(f32), 4).reshape(1, 128)
    Wu1b = _block_diag(Wu1.astype(f32), 4)
    bu1b = jnp.tile(bu1.astype(f32), 4).reshape(1, 128)
    Wu2b = _block_diag(Wu2.astype(f32), 4)
    bu2b = jnp.tile(bu2.astype(f32), 4).reshape(1, 128)

    A = _block_diag(Wr1[:D].astype(f32), 4)
    Bm = _block_diag(Wr1[D:].astype(f32), 4)
    br1q = jnp.tile(br1.astype(f32), 4).reshape(1, 192)
    C = _block_diag(Wr2.astype(f32), 4)
    br2q = jnp.tile(br2.astype(f32), 4).reshape(1, 4)

    h = _emb_gather(atoms_f, embed.astype(f32))
    deg4 = _degree(src_f).reshape(NPAD // 4, 128)

    for _ in range(3):
        gs, gd = _gather2(h, src_f, dst_f)
        act = _edge_mlp(gs.reshape(EPAD // 4, 128),
                        gd.reshape(EPAD // 4, 128), W1b, b1b)
        S = _scatter_add(act.reshape(EPAD, D), src_f)
        h4 = _node_update(h.reshape(NPAD // 4, 128),
                          S.reshape(NPAD // 4, 128), deg4,
                          W2b, b2b, Wu1b, bu1b, Wu2b, bu2b)
        h = h4.reshape(NPAD, D)

    y4 = _readout(h.reshape(NPAD // 4, 128), oth4, A, Bm, br1q, C, br2q)
    partials = _segment_sum(y4.reshape(NPAD), seg_f)
    return partials.reshape(NW, SEGW).sum(axis=0)[:nb]


# speedup vs baseline: 1.0899x; 1.0899x over previous
import functools

import jax
import jax.numpy as jnp
from jax import lax
from jax.experimental import pallas as pl
from jax.experimental.pallas import tpu as pltpu
from jax.experimental.pallas import tpu_sc as plsc

N = 100128
E = 1602048
D = 32
NW = 32
NPAD = 102400
NR_W = NPAD // NW
EPAD = 1605632
ER_W = EPAD // NW
ER_T = EPAD // 16
HALF = NPAD // 2
QR = NPAD // 4
SEGW = 464

_mesh_cache = []


def _MESH():
    if not _mesh_cache:
        _mesh_cache.append(plsc.VectorSubcoreMesh(
            core_axis_name="c", subcore_axis_name="s",
            num_cores=2, num_subcores=16))
    return _mesh_cache[0]


def _wid():
    return lax.axis_index("s") * 2 + lax.axis_index("c")


_SC_PARAMS = dict(use_tc_tiling_on_sc=False)


def _emb_gather(atoms_f, embed):
    @functools.partial(
        pl.kernel,
        out_type=jax.ShapeDtypeStruct((NPAD, D), jnp.float32),
        mesh=_MESH(),
        compiler_params=pltpu.CompilerParams(**_SC_PARAMS),
        scratch_types=[
            pltpu.VMEM((NR_W,), jnp.int32),
            pltpu.VMEM((NR_W, D), jnp.float32),
            pltpu.SemaphoreType.DMA,
        ],
    )
    def k(atoms_hbm, embed_hbm, out_hbm, idx_v, rows_v, sem):
        w = _wid()
        pltpu.sync_copy(atoms_hbm.at[pl.ds(w * NR_W, NR_W)], idx_v)
        pltpu.async_copy(embed_hbm.at[idx_v], rows_v, sem).wait()
        pltpu.sync_copy(rows_v, out_hbm.at[pl.ds(w * NR_W, NR_W)])

    return k(atoms_f, embed)


def _gather2(h, src_f, dst_f):
    CR = 896
    NCH = ER_W // CR

    @functools.partial(
        pl.kernel,
        out_type=[jax.ShapeDtypeStruct((EPAD, D), jnp.float32),
                  jax.ShapeDtypeStruct((EPAD, D), jnp.float32)],
        mesh=_MESH(),
        compiler_params=pltpu.CompilerParams(**_SC_PARAMS),
        scratch_types=[
            pltpu.VMEM((CR,), jnp.int32), pltpu.VMEM((CR,), jnp.int32),
            pltpu.VMEM((CR,), jnp.int32), pltpu.VMEM((CR,), jnp.int32),
            pltpu.VMEM((CR, D), jnp.float32), pltpu.VMEM((CR, D), jnp.float32),
            pltpu.VMEM((CR, D), jnp.float32), pltpu.VMEM((CR, D), jnp.float32),
            pltpu.SemaphoreType.DMA,
            pltpu.SemaphoreType.DMA,
        ],
    )
    def k(h_hbm, src_hbm, dst_hbm, gs_hbm, gd_hbm,
          ixs0, ixs1, ixd0, ixd1, rs0, rs1, rd0, rd1, gsem, wsem):
        w = _wid()
        bufs = [(ixs0, ixd0, rs0, rd0), (ixs1, ixd1, rs1, rd1)]

        def chunk(ci, p, drain):
            ixs, ixd, rs, rd = bufs[p]
            e0 = w * ER_W + ci * CR
            pltpu.sync_copy(src_hbm.at[pl.ds(e0, CR)], ixs)
            pltpu.sync_copy(dst_hbm.at[pl.ds(e0, CR)], ixd)
            if drain:
                pltpu.make_async_copy(gs_hbm.at[pl.ds(0, CR)], rs, wsem).wait()
                pltpu.make_async_copy(gs_hbm.at[pl.ds(0, CR)], rd, wsem).wait()
            g1 = pltpu.async_copy(h_hbm.at[ixs], rs, gsem)
            g2 = pltpu.async_copy(h_hbm.at[ixd], rd, gsem)
            g1.wait()
            g2.wait()
            pltpu.async_copy(rs, gs_hbm.at[pl.ds(e0, CR)], wsem)
            pltpu.async_copy(rd, gd_hbm.at[pl.ds(e0, CR)], wsem)

        chunk(0, 0, False)
        chunk(1, 1, False)

        @pl.loop(2, NCH, step=2)
        def _loop(base):
            chunk(base, 0, True)
            chunk(base + 1, 1, True)

        for p in range(2):
            _, _, rs, rd = bufs[p]
            pltpu.make_async_copy(gs_hbm.at[pl.ds(0, CR)], rs, wsem).wait()
            pltpu.make_async_copy(gs_hbm.at[pl.ds(0, CR)], rd, wsem).wait()

    return k(h, src_f, dst_f)


def _scatter_part(act, src_f, kbase):
    CR = 2048
    NCH = ER_T // CR
    ACC = QR + 256
    TR = ACC // 16
    WB = 800

    @functools.partial(
        pl.kernel,
        out_type=jax.ShapeDtypeStruct((2 * QR, D), jnp.float32),
        mesh=_MESH(),
        compiler_params=pltpu.CompilerParams(**_SC_PARAMS),
        scratch_types=[
            pltpu.VMEM((CR,), jnp.int32),
            pltpu.VMEM((CR, D), jnp.float32),
            pltpu.VMEM_SHARED((ACC, D), jnp.float32),
            pltpu.SemaphoreType.DMA,
        ],
    )
    def k(act_hbm, src_hbm, out_hbm, ix, vv, acc_sh, asem):
        c = lax.axis_index("c")
        s = lax.axis_index("s")
        base = kbase + c * QR
        zero16 = jnp.zeros((16,), jnp.float32)

        def zrow(r, carry):
            vv[r, pl.ds(0, 16)] = zero16
            vv[r, pl.ds(16, 16)] = zero16
            return carry

        lax.fori_loop(0, TR, zrow, 0)
        pltpu.sync_copy(vv.at[pl.ds(0, TR)], acc_sh.at[pl.ds(s * TR, TR)])
        plsc.subcore_barrier()

        def chunk(ci, carry):
            e0 = (s * NCH + ci) * CR
            pltpu.sync_copy(src_hbm.at[pl.ds(e0, CR)], ix)
            pltpu.sync_copy(act_hbm.at[pl.ds(e0, CR)], vv)

            def lrow(t, c2):
                v = ix[pl.ds(t * 16, 16)]
                ok = (v >= base) & (v < base + QR)
                ix[pl.ds(t * 16, 16)] = jnp.where(ok, v - base, QR)
                return c2

            lax.fori_loop(0, CR // 16, lrow, 0)
            pltpu.async_copy(vv, acc_sh.at[ix], asem, add=True).wait()
            return carry

        lax.fori_loop(0, NCH, chunk, 0)
        plsc.subcore_barrier()

        for t in range(2):
            pltpu.sync_copy(acc_sh.at[pl.ds(s * (2 * WB) + t * WB, WB)],
                            vv.at[pl.ds(0, WB)])
            pltpu.sync_copy(vv.at[pl.ds(0, WB)],
                            out_hbm.at[pl.ds(c * QR + s * (2 * WB) + t * WB, WB)])

    return k(act, src_f)


def _scatter_add(act, src_f):
    lo = _scatter_part(act, src_f, 0)
    hi = _scatter_part(act, src_f, HALF)
    return jnp.concatenate([lo, hi], axis=0)


def _degree_part(src_f, kbase):
    CR = 1024
    NCH = ER_T // CR
    ACC = QR + 256
    TR = ACC // 16
    WB = 800

    @functools.partial(
        pl.kernel,
        out_type=jax.ShapeDtypeStruct((2 * QR, D), jnp.float32),
        mesh=_MESH(),
        compiler_params=pltpu.CompilerParams(**_SC_PARAMS),
        scratch_types=[
            pltpu.VMEM((CR,), jnp.int32), pltpu.VMEM((CR,), jnp.int32),
            pltpu.VMEM((CR, D), jnp.float32),
            pltpu.VMEM_SHARED((ACC, D), jnp.float32),
            pltpu.SemaphoreType.DMA,
        ],
    )
    def k(src_hbm, out_hbm, ix0, ix1, ones_v, acc_sh, asem):
        c = lax.axis_index("c")
        s = lax.axis_index("s")
        base = kbase + c * QR
        zero16 = jnp.zeros((16,), jnp.float32)
        one16 = jnp.ones((16,), jnp.float32)
        ixbufs = [ix0, ix1]

        def zrow(r, carry):
            ones_v[r, pl.ds(0, 16)] = zero16
            ones_v[r, pl.ds(16, 16)] = zero16
            return carry

        lax.fori_loop(0, CR, zrow, 0)
        pltpu.sync_copy(ones_v, acc_sh.at[pl.ds(s * TR, CR)])
        pltpu.sync_copy(ones_v.at[pl.ds(0, TR - CR)],
                        acc_sh.at[pl.ds(s * TR + CR, TR - CR)])

        def orow(r, carry):
            ones_v[r, pl.ds(0, 16)] = one16
            ones_v[r, pl.ds(16, 16)] = one16
            return carry

        lax.fori_loop(0, CR, orow, 0)
        plsc.subcore_barrier()

        def chunk(ci, p, drain):
            ix = ixbufs[p]
            e0 = s * ER_T + ci * CR
            if drain:
                pltpu.make_async_copy(out_hbm.at[pl.ds(0, CR)], ones_v,
                                      asem).wait()
            pltpu.sync_copy(src_hbm.at[pl.ds(e0, CR)], ix)

            def lrow(t, c2):
                v = ix[pl.ds(t * 16, 16)]
                ok = (v >= base) & (v < base + QR)
                ix[pl.ds(t * 16, 16)] = jnp.where(ok, v - base, QR)
                return c2

            lax.fori_loop(0, CR // 16, lrow, 0)
            pltpu.async_copy(ones_v, acc_sh.at[ix], asem, add=True)

        chunk(0, 0, False)
        chunk(1, 1, False)

        @pl.loop(2, NCH, step=2)
        def _loop(b2):
            chunk(b2, 0, True)
            chunk(b2 + 1, 1, True)

        for _p in range(2):
            pltpu.make_async_copy(out_hbm.at[pl.ds(0, CR)], ones_v,
                                  asem).wait()
        plsc.subcore_barrier()

        for t in range(2):
            pltpu.sync_copy(acc_sh.at[pl.ds(s * (2 * WB) + t * WB, WB)],
                            ones_v.at[pl.ds(0, WB)])
            pltpu.sync_copy(ones_v.at[pl.ds(0, WB)],
                            out_hbm.at[pl.ds(c * QR + s * (2 * WB) + t * WB, WB)])

    return k(src_f)


def _degree(src_f):
    lo = _degree_part(src_f, 0)
    hi = _degree_part(src_f, HALF)
    return jnp.concatenate([lo, hi], axis=0)


def _segment_sum(y_f, seg_f):
    PER_LANE = NR_W // 16

    @functools.partial(
        pl.kernel,
        out_type=jax.ShapeDtypeStruct((NW * SEGW,), jnp.float32),
        mesh=_MESH(),
        compiler_params=pltpu.CompilerParams(
            needs_layout_passes=False, **_SC_PARAMS),
        scratch_types=[
            pltpu.VMEM((NR_W,), jnp.float32),
            pltpu.VMEM((NR_W,), jnp.int32),
            pltpu.VMEM((16, SEGW), jnp.float32),
            pltpu.VMEM((SEGW,), jnp.float32),
        ],
    )
    def k(y_hbm, seg_hbm, out_hbm, y_v, seg_v, acc_v, pout_v):
        w = _wid()
        pltpu.sync_copy(y_hbm.at[pl.ds(w * NR_W, NR_W)], y_v)
        pltpu.sync_copy(seg_hbm.at[pl.ds(w * NR_W, NR_W)], seg_v)

        zero16 = jnp.zeros((16,), jnp.float32)

        def zacc(kk, carry):
            for r in range(16):
                acc_v[r, pl.ds(kk * 16, 16)] = zero16
            return carry

        lax.fori_loop(0, SEGW // 16, zacc, 0)

        lane = lax.iota(jnp.int32, 16)
        gbase = lane * PER_LANE

        def step(j, carry):
            cur, acc = carry
            g = gbase + j
            yv = plsc.load_gather(y_v, [g])
            sg = plsc.load_gather(seg_v, [g])
            changed = sg != cur
            plsc.addupdate_scatter(acc_v, [lane, cur], acc, mask=changed)
            acc = jnp.where(changed, yv, acc + yv)
            return sg, acc

        cur0 = jnp.full((16,), 460, jnp.int32)
        cur, acc = lax.fori_loop(0, PER_LANE, step, (cur0, zero16))
        plsc.addupdate_scatter(acc_v, [lane, cur], acc)

        def red(kk, carry):
            t = acc_v[0, pl.ds(kk * 16, 16)]
            for r in range(1, 16):
                t = t + acc_v[r, pl.ds(kk * 16, 16)]
            pout_v[pl.ds(kk * 16, 16)] = t
            return carry

        lax.fori_loop(0, SEGW // 16, red, 0)
        pltpu.sync_copy(pout_v, out_hbm.at[pl.ds(w * SEGW, SEGW)])

    return k(y_f, seg_f)


def _block_diag(w, copies):
    ki, ko = w.shape
    out = jnp.zeros((copies * ki, copies * ko), w.dtype)
    for i in range(copies):
        out = out.at[i * ki:(i + 1) * ki, i * ko:(i + 1) * ko].set(w)
    return out


def _edge_mlp(gs4, gd4, W1b, b1b):
    BLK = 2048
    R = EPAD // 4
    grid = (R // BLK,)

    def body(xs_ref, xd_ref, w_ref, b_ref, o_ref):
        x = xs_ref[...] * xd_ref[...]
        y = jnp.dot(x, w_ref[...], preferred_element_type=jnp.float32)
        o_ref[...] = jnp.maximum(y + b_ref[...], 0.0)

    return pl.pallas_call(
        body,
        grid=grid,
        in_specs=[
            pl.BlockSpec((BLK, 128), lambda i: (i, 0)),
            pl.BlockSpec((BLK, 128), lambda i: (i, 0)),
            pl.BlockSpec((128, 128), lambda i: (0, 0)),
            pl.BlockSpec((1, 128), lambda i: (0, 0)),
        ],
        out_specs=pl.BlockSpec((BLK, 128), lambda i: (i, 0)),
        out_shape=jax.ShapeDtypeStruct((R, 128), jnp.float32),
    )(gs4, gd4, W1b, b1b)


def _node_update(h4, S4, deg4, W2b, b2b, Wu1b, bu1b, Wu2b, bu2b):
    BLK = 1024
    R = NPAD // 4
    grid = (R // BLK,)

    def body(h_ref, s_ref, d_ref, w2_ref, b2_ref, wu1_ref, bu1_ref,
             wu2_ref, bu2_ref, o_ref):
        nm = jnp.dot(s_ref[...], w2_ref[...],
                     preferred_element_type=jnp.float32)
        nm = nm + d_ref[...] * b2_ref[...]
        t = jnp.maximum(
            jnp.dot(nm, wu1_ref[...], preferred_element_type=jnp.float32)
            + bu1_ref[...], 0.0)
        o_ref[...] = (h_ref[...]
                      + jnp.dot(t, wu2_ref[...],
                                preferred_element_type=jnp.float32)
                      + bu2_ref[...])

    wspec = pl.BlockSpec((128, 128), lambda i: (0, 0))
    bspec = pl.BlockSpec((1, 128), lambda i: (0, 0))
    return pl.pallas_call(
        body,
        grid=grid,
        in_specs=[
            pl.BlockSpec((BLK, 128), lambda i: (i, 0)),
            pl.BlockSpec((BLK, 128), lambda i: (i, 0)),
            pl.BlockSpec((BLK, 128), lambda i: (i, 0)),
            wspec, bspec, wspec, bspec, wspec, bspec,
        ],
        out_specs=pl.BlockSpec((BLK, 128), lambda i: (i, 0)),
        out_shape=jax.ShapeDtypeStruct((R, 128), jnp.float32),
    )(h4, S4, deg4, W2b, b2b, Wu1b, bu1b, Wu2b, bu2b)


def _readout(h4, oth4, A, Bm, br1q, C, br2q):
    BLK = 1024
    R = NPAD // 4
    grid = (R // BLK,)

    def body(h_ref, o_ref, a_ref, b_ref, br1_ref, c_ref, br2_ref, out_ref):
        y = (jnp.dot(h_ref[...], a_ref[...], preferred_element_type=jnp.float32)
             + jnp.dot(o_ref[...], b_ref[...], preferred_element_type=jnp.float32)
             + br1_ref[...])
        y = jnp.maximum(y, 0.0)
        out_ref[...] = (jnp.dot(y, c_ref[...], preferred_element_type=jnp.float32)
                        + br2_ref[...])

    return pl.pallas_call(
        body,
        grid=grid,
        in_specs=[
            pl.BlockSpec((BLK, 128), lambda i: (i, 0)),
            pl.BlockSpec((BLK, 64), lambda i: (i, 0)),
            pl.BlockSpec((128, 192), lambda i: (0, 0)),
            pl.BlockSpec((64, 192), lambda i: (0, 0)),
            pl.BlockSpec((1, 192), lambda i: (0, 0)),
            pl.BlockSpec((192, 4), lambda i: (0, 0)),
            pl.BlockSpec((1, 4), lambda i: (0, 0)),
        ],
        out_specs=pl.BlockSpec((BLK, 4), lambda i: (i, 0)),
        out_shape=jax.ShapeDtypeStruct((R, 4), jnp.float32),
    )(h4, oth4, A, Bm, br1q, C, br2q)


def kernel(encoded_atoms, edges, natoms, other_features, embed,
           Wm1, bm1, Wm2, bm2, Wu1, bu1, Wu2, bu2, Wr1, br1, Wr2, br2):
    f32 = jnp.float32
    atoms_f = jnp.pad(encoded_atoms.astype(jnp.int32), (0, NPAD - N))
    src_f = jnp.pad(edges[0].astype(jnp.int32), (0, EPAD - E),
                    constant_values=N)
    dst_f = jnp.pad(edges[1].astype(jnp.int32), (0, EPAD - E),
                    constant_values=N)

    nb = natoms.shape[0]
    seg = jnp.repeat(jnp.arange(nb, dtype=jnp.int32), natoms,
                     total_repeat_length=N)
    seg_f = jnp.pad(seg, (0, NPAD - N), constant_values=450)

    oth = jnp.pad(other_features.astype(f32), ((0, NPAD - N), (0, 0)))
    oth4 = oth.reshape(NPAD // 4, 64)

    W1b = _block_diag(Wm1.astype(f32), 4)
    b1b = jnp.tile(bm1.astype(f32), 4).reshape(1, 128)
    W2b = _block_diag(Wm2.astype(f32), 4)
    b2b = jnp.tile(bm2.astype(f32), 4).reshape(1, 128)
    Wu1b = _block_diag(Wu1.astype(f32), 4)
    bu1b = jnp.tile(bu1.astype(f32), 4).reshape(1, 128)
    Wu2b = _block_diag(Wu2.astype(f32), 4)
    bu2b = jnp.tile(bu2.astype(f32), 4).reshape(1, 128)

    A = _block_diag(Wr1[:D].astype(f32), 4)
    Bm = _block_diag(Wr1[D:].astype(f32), 4)
    br1q = jnp.tile(br1.astype(f32), 4).reshape(1, 192)
    C = _block_diag(Wr2.astype(f32), 4)
    br2q = jnp.tile(br2.astype(f32), 4).reshape(1, 4)

    h = _emb_gather(atoms_f, embed.astype(f32))
    deg4 = _degree(src_f).reshape(NPAD // 4, 128)

    for _ in range(3):
        gs, gd = _gather2(h, src_f, dst_f)
        act = _edge_mlp(gs.reshape(EPAD // 4, 128),
                        gd.reshape(EPAD // 4, 128), W1b, b1b)
        S = _scatter_add(act.reshape(EPAD, D), src_f)
        h4 = _node_update(h.reshape(NPAD // 4, 128),
                          S.reshape(NPAD // 4, 128), deg4,
                          W2b, b2b, Wu1b, bu1b, Wu2b, bu2b)
        h = h4.reshape(NPAD, D)

    y4 = _readout(h.reshape(NPAD // 4, 128), oth4, A, Bm, br1q, C, br2q)
    partials = _segment_sum(y4.reshape(NPAD), seg_f)
    return partials.reshape(NW, SEGW).sum(axis=0)[:nb]


# speedup vs baseline: 1.3280x; 1.2185x over previous
import functools

import jax
import jax.numpy as jnp
from jax import lax
from jax.experimental import pallas as pl
from jax.experimental.pallas import tpu as pltpu
from jax.experimental.pallas import tpu_sc as plsc

N = 100128
E = 1602048
D = 32
NW = 32
NPAD = 102400
NR_W = NPAD // NW
EPAD = 1605632
ER_W = EPAD // NW
ER_T = EPAD // 16
HALF = NPAD // 2
QR = NPAD // 4
SEGW = 464

_mesh_cache = []


def _MESH():
    if not _mesh_cache:
        _mesh_cache.append(plsc.VectorSubcoreMesh(
            core_axis_name="c", subcore_axis_name="s",
            num_cores=2, num_subcores=16))
    return _mesh_cache[0]


def _wid():
    return lax.axis_index("s") * 2 + lax.axis_index("c")


_SC_PARAMS = dict(use_tc_tiling_on_sc=False)


def _emb_gather(atoms_f, embed):
    @functools.partial(
        pl.kernel,
        out_type=jax.ShapeDtypeStruct((NPAD, D), jnp.float32),
        mesh=_MESH(),
        compiler_params=pltpu.CompilerParams(**_SC_PARAMS),
        scratch_types=[
            pltpu.VMEM((NR_W,), jnp.int32),
            pltpu.VMEM((NR_W, D), jnp.float32),
            pltpu.SemaphoreType.DMA,
        ],
    )
    def k(atoms_hbm, embed_hbm, out_hbm, idx_v, rows_v, sem):
        w = _wid()
        pltpu.sync_copy(atoms_hbm.at[pl.ds(w * NR_W, NR_W)], idx_v)
        pltpu.async_copy(embed_hbm.at[idx_v], rows_v, sem).wait()
        pltpu.sync_copy(rows_v, out_hbm.at[pl.ds(w * NR_W, NR_W)])

    return k(atoms_f, embed)


def _gather2(h, src_f, dst_f):
    CR = 896
    NCH = ER_W // CR

    @functools.partial(
        pl.kernel,
        out_type=[jax.ShapeDtypeStruct((EPAD, D), jnp.float32),
                  jax.ShapeDtypeStruct((EPAD, D), jnp.float32)],
        mesh=_MESH(),
        compiler_params=pltpu.CompilerParams(**_SC_PARAMS),
        scratch_types=[
            pltpu.VMEM((CR,), jnp.int32), pltpu.VMEM((CR,), jnp.int32),
            pltpu.VMEM((CR,), jnp.int32), pltpu.VMEM((CR,), jnp.int32),
            pltpu.VMEM((CR, D), jnp.float32), pltpu.VMEM((CR, D), jnp.float32),
            pltpu.VMEM((CR, D), jnp.float32), pltpu.VMEM((CR, D), jnp.float32),
            pltpu.SemaphoreType.DMA,
            pltpu.SemaphoreType.DMA,
        ],
    )
    def k(h_hbm, src_hbm, dst_hbm, gs_hbm, gd_hbm,
          ixs0, ixs1, ixd0, ixd1, rs0, rs1, rd0, rd1, gsem, wsem):
        w = _wid()
        bufs = [(ixs0, ixd0, rs0, rd0), (ixs1, ixd1, rs1, rd1)]

        def chunk(ci, p, drain):
            ixs, ixd, rs, rd = bufs[p]
            e0 = w * ER_W + ci * CR
            pltpu.sync_copy(src_hbm.at[pl.ds(e0, CR)], ixs)
            pltpu.sync_copy(dst_hbm.at[pl.ds(e0, CR)], ixd)
            if drain:
                pltpu.make_async_copy(gs_hbm.at[pl.ds(0, CR)], rs, wsem).wait()
                pltpu.make_async_copy(gs_hbm.at[pl.ds(0, CR)], rd, wsem).wait()
            g1 = pltpu.async_copy(h_hbm.at[ixs], rs, gsem)
            g2 = pltpu.async_copy(h_hbm.at[ixd], rd, gsem)
            g1.wait()
            g2.wait()
            pltpu.async_copy(rs, gs_hbm.at[pl.ds(e0, CR)], wsem)
            pltpu.async_copy(rd, gd_hbm.at[pl.ds(e0, CR)], wsem)

        chunk(0, 0, False)
        chunk(1, 1, False)

        @pl.loop(2, NCH, step=2)
        def _loop(base):
            chunk(base, 0, True)
            chunk(base + 1, 1, True)

        for p in range(2):
            _, _, rs, rd = bufs[p]
            pltpu.make_async_copy(gs_hbm.at[pl.ds(0, CR)], rs, wsem).wait()
            pltpu.make_async_copy(gs_hbm.at[pl.ds(0, CR)], rd, wsem).wait()

    return k(h, src_f, dst_f)


def _scatter_part(act, src_f, kbase):
    CR = 2048
    NCH = ER_T // CR
    ACC = QR + 256
    TR = ACC // 16
    WB = 800

    @functools.partial(
        pl.kernel,
        out_type=jax.ShapeDtypeStruct((2 * QR, D), jnp.float32),
        mesh=_MESH(),
        compiler_params=pltpu.CompilerParams(**_SC_PARAMS),
        scratch_types=[
            pltpu.VMEM((CR,), jnp.int32),
            pltpu.VMEM((CR, D), jnp.float32),
            pltpu.VMEM_SHARED((ACC, D), jnp.float32),
            pltpu.SemaphoreType.DMA,
        ],
    )
    def k(act_hbm, src_hbm, out_hbm, ix, vv, acc_sh, asem):
        c = lax.axis_index("c")
        s = lax.axis_index("s")
        base = kbase + c * QR
        zero16 = jnp.zeros((16,), jnp.float32)

        def zrow(r, carry):
            vv[r, pl.ds(0, 16)] = zero16
            vv[r, pl.ds(16, 16)] = zero16
            return carry

        lax.fori_loop(0, TR, zrow, 0)
        pltpu.sync_copy(vv.at[pl.ds(0, TR)], acc_sh.at[pl.ds(s * TR, TR)])
        plsc.subcore_barrier()

        def chunk(ci, carry):
            e0 = (s * NCH + ci) * CR
            pltpu.sync_copy(src_hbm.at[pl.ds(e0, CR)], ix)
            pltpu.sync_copy(act_hbm.at[pl.ds(e0, CR)], vv)

            def lrow(t, c2):
                v = ix[pl.ds(t * 16, 16)]
                ok = (v >= base) & (v < base + QR)
                ix[pl.ds(t * 16, 16)] = jnp.where(ok, v - base, QR)
                return c2

            lax.fori_loop(0, CR // 16, lrow, 0)
            pltpu.async_copy(vv, acc_sh.at[ix], asem, add=True).wait()
            return carry

        lax.fori_loop(0, NCH, chunk, 0)
        plsc.subcore_barrier()

        for t in range(2):
            pltpu.sync_copy(acc_sh.at[pl.ds(s * (2 * WB) + t * WB, WB)],
                            vv.at[pl.ds(0, WB)])
            pltpu.sync_copy(vv.at[pl.ds(0, WB)],
                            out_hbm.at[pl.ds(c * QR + s * (2 * WB) + t * WB, WB)])

    return k(act, src_f)


def _scatter_add(act, src_f):
    lo = _scatter_part(act, src_f, 0)
    hi = _scatter_part(act, src_f, HALF)
    return jnp.concatenate([lo, hi], axis=0)


def _degree_part(src_f, kbase):
    CR = 2048
    NCH = ER_T // CR
    ACC = QR + 256
    TR = ACC // 16
    WB = 800
    CPAD = CR + 160

    @functools.partial(
        pl.kernel,
        out_type=jax.ShapeDtypeStruct((2 * QR, D), jnp.float32),
        mesh=_MESH(),
        compiler_params=pltpu.CompilerParams(
            needs_layout_passes=False, **_SC_PARAMS),
        scratch_types=[
            pltpu.VMEM((CR,), jnp.int32),
            pltpu.VMEM((CPAD,), jnp.int32),
            pltpu.VMEM((TR, D), jnp.float32),
            pltpu.VMEM_SHARED((ACC, D), jnp.float32),
            pltpu.SemaphoreType.DMA,
        ],
    )
    def k(src_hbm, out_hbm, ix, cl, ones_v, acc_sh, asem):
        c = lax.axis_index("c")
        s = lax.axis_index("s")
        base = kbase + c * QR
        zero16 = jnp.zeros((16,), jnp.float32)
        one16 = jnp.ones((16,), jnp.float32)
        dump16 = jnp.full((16,), QR, jnp.int32)

        def zrow(r, carry):
            ones_v[r, pl.ds(0, 16)] = zero16
            ones_v[r, pl.ds(16, 16)] = zero16
            return carry

        lax.fori_loop(0, TR, zrow, 0)
        pltpu.sync_copy(ones_v, acc_sh.at[pl.ds(s * TR, TR)])

        def orow(r, carry):
            ones_v[r, pl.ds(0, 16)] = one16
            ones_v[r, pl.ds(16, 16)] = one16
            return carry

        lax.fori_loop(0, 128, orow, 0)
        plsc.subcore_barrier()

        def chunk(ci, carry):
            e0 = (s * NCH + ci) * CR
            pltpu.sync_copy(src_hbm.at[pl.ds(e0, CR)], ix)

            def sc16(t, cnt):
                v = ix[pl.ds(t * 16, 16)]
                ok = (v >= base) & (v < base + QR)
                plsc.store_compressed(cl.at[pl.ds(cnt, 16)], v - base,
                                      mask=ok)
                return cnt + jnp.sum(ok.astype(jnp.int32))

            cnt = lax.fori_loop(0, CR // 16, sc16, 0)
            for j in range(8):
                cl[pl.ds(cnt + j * 16, 16)] = dump16
            for j in range(16):
                @pl.when(j * 128 < cnt)
                def _():
                    pltpu.async_copy(
                        ones_v.at[pl.ds(0, 128)],
                        acc_sh.at[cl.at[pl.ds(j * 128, 128)]],
                        asem, add=True).wait()
            return carry

        lax.fori_loop(0, NCH, chunk, 0)
        plsc.subcore_barrier()

        for t in range(2):
            pltpu.sync_copy(acc_sh.at[pl.ds(s * (2 * WB) + t * WB, WB)],
                            ones_v.at[pl.ds(0, WB)])
            pltpu.sync_copy(ones_v.at[pl.ds(0, WB)],
                            out_hbm.at[pl.ds(c * QR + s * (2 * WB) + t * WB, WB)])

    return k(src_f)


def _degree(src_f):
    lo = _degree_part(src_f, 0)
    hi = _degree_part(src_f, HALF)
    return jnp.concatenate([lo, hi], axis=0)


def _segment_sum(y_f, seg_f):
    PER_LANE = NR_W // 16

    @functools.partial(
        pl.kernel,
        out_type=jax.ShapeDtypeStruct((NW * SEGW,), jnp.float32),
        mesh=_MESH(),
        compiler_params=pltpu.CompilerParams(
            needs_layout_passes=False, **_SC_PARAMS),
        scratch_types=[
            pltpu.VMEM((NR_W,), jnp.float32),
            pltpu.VMEM((NR_W,), jnp.int32),
            pltpu.VMEM((16, SEGW), jnp.float32),
            pltpu.VMEM((SEGW,), jnp.float32),
        ],
    )
    def k(y_hbm, seg_hbm, out_hbm, y_v, seg_v, acc_v, pout_v):
        w = _wid()
        pltpu.sync_copy(y_hbm.at[pl.ds(w * NR_W, NR_W)], y_v)
        pltpu.sync_copy(seg_hbm.at[pl.ds(w * NR_W, NR_W)], seg_v)

        zero16 = jnp.zeros((16,), jnp.float32)

        def zacc(kk, carry):
            for r in range(16):
                acc_v[r, pl.ds(kk * 16, 16)] = zero16
            return carry

        lax.fori_loop(0, SEGW // 16, zacc, 0)

        lane = lax.iota(jnp.int32, 16)
        gbase = lane * PER_LANE

        def step(j, carry):
            cur, acc = carry
            g = gbase + j
            yv = plsc.load_gather(y_v, [g])
            sg = plsc.load_gather(seg_v, [g])
            changed = sg != cur
            plsc.addupdate_scatter(acc_v, [lane, cur], acc, mask=changed)
            acc = jnp.where(changed, yv, acc + yv)
            return sg, acc

        cur0 = jnp.full((16,), 460, jnp.int32)
        cur, acc = lax.fori_loop(0, PER_LANE, step, (cur0, zero16))
        plsc.addupdate_scatter(acc_v, [lane, cur], acc)

        def red(kk, carry):
            t = acc_v[0, pl.ds(kk * 16, 16)]
            for r in range(1, 16):
                t = t + acc_v[r, pl.ds(kk * 16, 16)]
            pout_v[pl.ds(kk * 16, 16)] = t
            return carry

        lax.fori_loop(0, SEGW // 16, red, 0)
        pltpu.sync_copy(pout_v, out_hbm.at[pl.ds(w * SEGW, SEGW)])

    return k(y_f, seg_f)


def _block_diag(w, copies):
    ki, ko = w.shape
    out = jnp.zeros((copies * ki, copies * ko), w.dtype)
    for i in range(copies):
        out = out.at[i * ki:(i + 1) * ki, i * ko:(i + 1) * ko].set(w)
    return out


def _edge_mlp(gs4, gd4, W1b, b1b):
    BLK = 2048
    R = EPAD // 4
    grid = (R // BLK,)

    def body(xs_ref, xd_ref, w_ref, b_ref, o_ref):
        x = xs_ref[...] * xd_ref[...]
        y = jnp.dot(x, w_ref[...], preferred_element_type=jnp.float32)
        o_ref[...] = jnp.maximum(y + b_ref[...], 0.0)

    return pl.pallas_call(
        body,
        grid=grid,
        in_specs=[
            pl.BlockSpec((BLK, 128), lambda i: (i, 0)),
            pl.BlockSpec((BLK, 128), lambda i: (i, 0)),
            pl.BlockSpec((128, 128), lambda i: (0, 0)),
            pl.BlockSpec((1, 128), lambda i: (0, 0)),
        ],
        out_specs=pl.BlockSpec((BLK, 128), lambda i: (i, 0)),
        out_shape=jax.ShapeDtypeStruct((R, 128), jnp.float32),
    )(gs4, gd4, W1b, b1b)


def _node_update(h4, S4, deg4, W2b, b2b, Wu1b, bu1b, Wu2b, bu2b):
    BLK = 1024
    R = NPAD // 4
    grid = (R // BLK,)

    def body(h_ref, s_ref, d_ref, w2_ref, b2_ref, wu1_ref, bu1_ref,
             wu2_ref, bu2_ref, o_ref):
        nm = jnp.dot(s_ref[...], w2_ref[...],
                     preferred_element_type=jnp.float32)
        nm = nm + d_ref[...] * b2_ref[...]
        t = jnp.maximum(
            jnp.dot(nm, wu1_ref[...], preferred_element_type=jnp.float32)
            + bu1_ref[...], 0.0)
        o_ref[...] = (h_ref[...]
                      + jnp.dot(t, wu2_ref[...],
                                preferred_element_type=jnp.float32)
                      + bu2_ref[...])

    wspec = pl.BlockSpec((128, 128), lambda i: (0, 0))
    bspec = pl.BlockSpec((1, 128), lambda i: (0, 0))
    return pl.pallas_call(
        body,
        grid=grid,
        in_specs=[
            pl.BlockSpec((BLK, 128), lambda i: (i, 0)),
            pl.BlockSpec((BLK, 128), lambda i: (i, 0)),
            pl.BlockSpec((BLK, 128), lambda i: (i, 0)),
            wspec, bspec, wspec, bspec, wspec, bspec,
        ],
        out_specs=pl.BlockSpec((BLK, 128), lambda i: (i, 0)),
        out_shape=jax.ShapeDtypeStruct((R, 128), jnp.float32),
    )(h4, S4, deg4, W2b, b2b, Wu1b, bu1b, Wu2b, bu2b)


def _readout(h4, oth4, A, Bm, br1q, C, br2q):
    BLK = 1024
    R = NPAD // 4
    grid = (R // BLK,)

    def body(h_ref, o_ref, a_ref, b_ref, br1_ref, c_ref, br2_ref, out_ref):
        y = (jnp.dot(h_ref[...], a_ref[...], preferred_element_type=jnp.float32)
             + jnp.dot(o_ref[...], b_ref[...], preferred_element_type=jnp.float32)
             + br1_ref[...])
        y = jnp.maximum(y, 0.0)
        out_ref[...] = (jnp.dot(y, c_ref[...], preferred_element_type=jnp.float32)
                        + br2_ref[...])

    return pl.pallas_call(
        body,
        grid=grid,
        in_specs=[
            pl.BlockSpec((BLK, 128), lambda i: (i, 0)),
            pl.BlockSpec((BLK, 64), lambda i: (i, 0)),
            pl.BlockSpec((128, 192), lambda i: (0, 0)),
            pl.BlockSpec((64, 192), lambda i: (0, 0)),
            pl.BlockSpec((1, 192), lambda i: (0, 0)),
            pl.BlockSpec((192, 4), lambda i: (0, 0)),
            pl.BlockSpec((1, 4), lambda i: (0, 0)),
        ],
        out_specs=pl.BlockSpec((BLK, 4), lambda i: (i, 0)),
        out_shape=jax.ShapeDtypeStruct((R, 4), jnp.float32),
    )(h4, oth4, A, Bm, br1q, C, br2q)


def kernel(encoded_atoms, edges, natoms, other_features, embed,
           Wm1, bm1, Wm2, bm2, Wu1, bu1, Wu2, bu2, Wr1, br1, Wr2, br2):
    f32 = jnp.float32
    atoms_f = jnp.pad(encoded_atoms.astype(jnp.int32), (0, NPAD - N))
    src_f = jnp.pad(edges[0].astype(jnp.int32), (0, EPAD - E),
                    constant_values=N)
    dst_f = jnp.pad(edges[1].astype(jnp.int32), (0, EPAD - E),
                    constant_values=N)

    nb = natoms.shape[0]
    seg = jnp.repeat(jnp.arange(nb, dtype=jnp.int32), natoms,
                     total_repeat_length=N)
    seg_f = jnp.pad(seg, (0, NPAD - N), constant_values=450)

    oth = jnp.pad(other_features.astype(f32), ((0, NPAD - N), (0, 0)))
    oth4 = oth.reshape(NPAD // 4, 64)

    W1b = _block_diag(Wm1.astype(f32), 4)
    b1b = jnp.tile(bm1.astype(f32), 4).reshape(1, 128)
    W2b = _block_diag(Wm2.astype(f32), 4)
    b2b = jnp.tile(bm2.astype(f32), 4).reshape(1, 128)
    Wu1b = _block_diag(Wu1.astype(f32), 4)
    bu1b = jnp.tile(bu1.astype(f32), 4).reshape(1, 128)
    Wu2b = _block_diag(Wu2.astype(f32), 4)
    bu2b = jnp.tile(bu2.astype(f32), 4).reshape(1, 128)

    A = _block_diag(Wr1[:D].astype(f32), 4)
    Bm = _block_diag(Wr1[D:].astype(f32), 4)
    br1q = jnp.tile(br1.astype(f32), 4).reshape(1, 192)
    C = _block_diag(Wr2.astype(f32), 4)
    br2q = jnp.tile(br2.astype(f32), 4).reshape(1, 4)

    h = _emb_gather(atoms_f, embed.astype(f32))
    deg4 = _degree(src_f).reshape(NPAD // 4, 128)

    for _ in range(3):
        gs, gd = _gather2(h, src_f, dst_f)
        act = _edge_mlp(gs.reshape(EPAD // 4, 128),
                        gd.reshape(EPAD // 4, 128), W1b, b1b)
        S = _scatter_add(act.reshape(EPAD, D), src_f)
        h4 = _node_update(h.reshape(NPAD // 4, 128),
                          S.reshape(NPAD // 4, 128), deg4,
                          W2b, b2b, Wu1b, bu1b, Wu2b, bu2b)
        h = h4.reshape(NPAD, D)

    y4 = _readout(h.reshape(NPAD // 4, 128), oth4, A, Bm, br1q, C, br2q)
    partials = _segment_sum(y4.reshape(NPAD), seg_f)
    return partials.reshape(NW, SEGW).sum(axis=0)[:nb]


# speedup vs baseline: 1.5802x; 1.1899x over previous
import functools

import jax
import jax.numpy as jnp
from jax import lax
from jax.experimental import pallas as pl
from jax.experimental.pallas import tpu as pltpu
from jax.experimental.pallas import tpu_sc as plsc

N = 100128
E = 1602048
D = 32
NW = 32
NPAD = 102400
NR_W = NPAD // NW
EPAD = 1605632
ER_W = EPAD // NW
ER_T = EPAD // 16
HALF = NPAD // 2
QR = NPAD // 4
SEGW = 464

_mesh_cache = []


def _MESH():
    if not _mesh_cache:
        _mesh_cache.append(plsc.VectorSubcoreMesh(
            core_axis_name="c", subcore_axis_name="s",
            num_cores=2, num_subcores=16))
    return _mesh_cache[0]


def _wid():
    return lax.axis_index("s") * 2 + lax.axis_index("c")


_SC_PARAMS = dict(use_tc_tiling_on_sc=False)


def _emb_gather(atoms_f, embed):
    @functools.partial(
        pl.kernel,
        out_type=jax.ShapeDtypeStruct((NPAD, D), jnp.float32),
        mesh=_MESH(),
        compiler_params=pltpu.CompilerParams(**_SC_PARAMS),
        scratch_types=[
            pltpu.VMEM((NR_W,), jnp.int32),
            pltpu.VMEM((NR_W, D), jnp.float32),
            pltpu.SemaphoreType.DMA,
        ],
    )
    def k(atoms_hbm, embed_hbm, out_hbm, idx_v, rows_v, sem):
        w = _wid()
        pltpu.sync_copy(atoms_hbm.at[pl.ds(w * NR_W, NR_W)], idx_v)
        pltpu.async_copy(embed_hbm.at[idx_v], rows_v, sem).wait()
        pltpu.sync_copy(rows_v, out_hbm.at[pl.ds(w * NR_W, NR_W)])

    return k(atoms_f, embed)


def _gather2(h, src_f, dst_f):
    CR = 896
    NCH = ER_W // CR

    @functools.partial(
        pl.kernel,
        out_type=[jax.ShapeDtypeStruct((EPAD, D), jnp.float32),
                  jax.ShapeDtypeStruct((EPAD, D), jnp.float32)],
        mesh=_MESH(),
        compiler_params=pltpu.CompilerParams(**_SC_PARAMS),
        scratch_types=[
            pltpu.VMEM((CR,), jnp.int32), pltpu.VMEM((CR,), jnp.int32),
            pltpu.VMEM((CR,), jnp.int32), pltpu.VMEM((CR,), jnp.int32),
            pltpu.VMEM((CR, D), jnp.float32), pltpu.VMEM((CR, D), jnp.float32),
            pltpu.VMEM((CR, D), jnp.float32), pltpu.VMEM((CR, D), jnp.float32),
            pltpu.SemaphoreType.DMA,
            pltpu.SemaphoreType.DMA,
        ],
    )
    def k(h_hbm, src_hbm, dst_hbm, gs_hbm, gd_hbm,
          ixs0, ixs1, ixd0, ixd1, rs0, rs1, rd0, rd1, gsem, wsem):
        w = _wid()
        bufs = [(ixs0, ixd0, rs0, rd0), (ixs1, ixd1, rs1, rd1)]

        def chunk(ci, p, drain):
            ixs, ixd, rs, rd = bufs[p]
            e0 = w * ER_W + ci * CR
            pltpu.sync_copy(src_hbm.at[pl.ds(e0, CR)], ixs)
            pltpu.sync_copy(dst_hbm.at[pl.ds(e0, CR)], ixd)
            if drain:
                pltpu.make_async_copy(gs_hbm.at[pl.ds(0, CR)], rs, wsem).wait()
                pltpu.make_async_copy(gs_hbm.at[pl.ds(0, CR)], rd, wsem).wait()
            g1 = pltpu.async_copy(h_hbm.at[ixs], rs, gsem)
            g2 = pltpu.async_copy(h_hbm.at[ixd], rd, gsem)
            g1.wait()
            g2.wait()
            pltpu.async_copy(rs, gs_hbm.at[pl.ds(e0, CR)], wsem)
            pltpu.async_copy(rd, gd_hbm.at[pl.ds(e0, CR)], wsem)

        chunk(0, 0, False)
        chunk(1, 1, False)

        @pl.loop(2, NCH, step=2)
        def _loop(base):
            chunk(base, 0, True)
            chunk(base + 1, 1, True)

        for p in range(2):
            _, _, rs, rd = bufs[p]
            pltpu.make_async_copy(gs_hbm.at[pl.ds(0, CR)], rs, wsem).wait()
            pltpu.make_async_copy(gs_hbm.at[pl.ds(0, CR)], rd, wsem).wait()

    return k(h, src_f, dst_f)


def _scatter_part(act, src_f, kbase):
    CR = 2048
    NCH = ER_T // CR
    ACC = QR + 256
    TR = ACC // 16
    WB = 800
    CPAD = CR + 160

    @functools.partial(
        pl.kernel,
        out_type=jax.ShapeDtypeStruct((2 * QR, D), jnp.float32),
        mesh=_MESH(),
        compiler_params=pltpu.CompilerParams(
            needs_layout_passes=False, **_SC_PARAMS),
        scratch_types=[
            pltpu.VMEM((CR,), jnp.int32),
            pltpu.VMEM((CPAD,), jnp.int32),
            pltpu.VMEM((CPAD,), jnp.int32),
            pltpu.VMEM((TR, D), jnp.float32),
            pltpu.VMEM_SHARED((ACC, D), jnp.float32),
            pltpu.SemaphoreType.DMA,
        ],
    )
    def k(act_hbm, src_hbm, out_hbm, ix, ce, cl, vv, acc_sh, asem):
        c = lax.axis_index("c")
        s = lax.axis_index("s")
        base = kbase + c * QR
        zero16 = jnp.zeros((16,), jnp.float32)
        zero16i = jnp.zeros((16,), jnp.int32)
        dump16 = jnp.full((16,), QR, jnp.int32)
        iota16 = lax.iota(jnp.int32, 16)

        def zrow(r, carry):
            vv[r, pl.ds(0, 16)] = zero16
            vv[r, pl.ds(16, 16)] = zero16
            return carry

        lax.fori_loop(0, TR, zrow, 0)
        pltpu.sync_copy(vv, acc_sh.at[pl.ds(s * TR, TR)])
        plsc.subcore_barrier()

        def chunk(ci, carry):
            e0 = (s * NCH + ci) * CR
            pltpu.sync_copy(src_hbm.at[pl.ds(e0, CR)], ix)

            def sc16(t, cnt):
                v = ix[pl.ds(t * 16, 16)]
                ok = (v >= base) & (v < base + QR)
                plsc.store_compressed(ce.at[pl.ds(cnt, 16)],
                                      iota16 + (e0 + t * 16), mask=ok)
                plsc.store_compressed(cl.at[pl.ds(cnt, 16)], v - base,
                                      mask=ok)
                return cnt + jnp.sum(ok.astype(jnp.int32))

            cnt = lax.fori_loop(0, CR // 16, sc16, 0)
            for j in range(8):
                ce[pl.ds(cnt + j * 16, 16)] = zero16i
                cl[pl.ds(cnt + j * 16, 16)] = dump16
            for j in range(16):
                @pl.when(j * 128 < cnt)
                def _():
                    pltpu.async_copy(
                        act_hbm.at[ce.at[pl.ds(j * 128, 128)]],
                        vv.at[pl.ds(0, 128)], asem).wait()
                    pltpu.async_copy(
                        vv.at[pl.ds(0, 128)],
                        acc_sh.at[cl.at[pl.ds(j * 128, 128)]],
                        asem, add=True).wait()
            return carry

        lax.fori_loop(0, NCH, chunk, 0)
        plsc.subcore_barrier()

        for t in range(2):
            pltpu.sync_copy(acc_sh.at[pl.ds(s * (2 * WB) + t * WB, WB)],
                            vv.at[pl.ds(0, WB)])
            pltpu.sync_copy(vv.at[pl.ds(0, WB)],
                            out_hbm.at[pl.ds(c * QR + s * (2 * WB) + t * WB, WB)])

    return k(act, src_f)


def _scatter_add(act, src_f):
    lo = _scatter_part(act, src_f, 0)
    hi = _scatter_part(act, src_f, HALF)
    return jnp.concatenate([lo, hi], axis=0)


def _degree_part(src_f, kbase):
    CR = 2048
    NCH = ER_T // CR
    ACC = QR + 256
    TR = ACC // 16
    WB = 800
    CPAD = CR + 160

    @functools.partial(
        pl.kernel,
        out_type=jax.ShapeDtypeStruct((2 * QR, D), jnp.float32),
        mesh=_MESH(),
        compiler_params=pltpu.CompilerParams(
            needs_layout_passes=False, **_SC_PARAMS),
        scratch_types=[
            pltpu.VMEM((CR,), jnp.int32),
            pltpu.VMEM((CPAD,), jnp.int32),
            pltpu.VMEM((TR, D), jnp.float32),
            pltpu.VMEM_SHARED((ACC, D), jnp.float32),
            pltpu.SemaphoreType.DMA,
        ],
    )
    def k(src_hbm, out_hbm, ix, cl, ones_v, acc_sh, asem):
        c = lax.axis_index("c")
        s = lax.axis_index("s")
        base = kbase + c * QR
        zero16 = jnp.zeros((16,), jnp.float32)
        one16 = jnp.ones((16,), jnp.float32)
        dump16 = jnp.full((16,), QR, jnp.int32)

        def zrow(r, carry):
            ones_v[r, pl.ds(0, 16)] = zero16
            ones_v[r, pl.ds(16, 16)] = zero16
            return carry

        lax.fori_loop(0, TR, zrow, 0)
        pltpu.sync_copy(ones_v, acc_sh.at[pl.ds(s * TR, TR)])

        def orow(r, carry):
            ones_v[r, pl.ds(0, 16)] = one16
            ones_v[r, pl.ds(16, 16)] = one16
            return carry

        lax.fori_loop(0, 128, orow, 0)
        plsc.subcore_barrier()

        def chunk(ci, carry):
            e0 = (s * NCH + ci) * CR
            pltpu.sync_copy(src_hbm.at[pl.ds(e0, CR)], ix)

            def sc16(t, cnt):
                v = ix[pl.ds(t * 16, 16)]
                ok = (v >= base) & (v < base + QR)
                plsc.store_compressed(cl.at[pl.ds(cnt, 16)], v - base,
                                      mask=ok)
                return cnt + jnp.sum(ok.astype(jnp.int32))

            cnt = lax.fori_loop(0, CR // 16, sc16, 0)
            for j in range(8):
                cl[pl.ds(cnt + j * 16, 16)] = dump16
            for j in range(16):
                @pl.when(j * 128 < cnt)
                def _():
                    pltpu.async_copy(
                        ones_v.at[pl.ds(0, 128)],
                        acc_sh.at[cl.at[pl.ds(j * 128, 128)]],
                        asem, add=True).wait()
            return carry

        lax.fori_loop(0, NCH, chunk, 0)
        plsc.subcore_barrier()

        for t in range(2):
            pltpu.sync_copy(acc_sh.at[pl.ds(s * (2 * WB) + t * WB, WB)],
                            ones_v.at[pl.ds(0, WB)])
            pltpu.sync_copy(ones_v.at[pl.ds(0, WB)],
                            out_hbm.at[pl.ds(c * QR + s * (2 * WB) + t * WB, WB)])

    return k(src_f)


def _degree(src_f):
    lo = _degree_part(src_f, 0)
    hi = _degree_part(src_f, HALF)
    return jnp.concatenate([lo, hi], axis=0)


def _segment_sum(y_f, seg_f):
    PER_LANE = NR_W // 16

    @functools.partial(
        pl.kernel,
        out_type=jax.ShapeDtypeStruct((NW * SEGW,), jnp.float32),
        mesh=_MESH(),
        compiler_params=pltpu.CompilerParams(
            needs_layout_passes=False, **_SC_PARAMS),
        scratch_types=[
            pltpu.VMEM((NR_W,), jnp.float32),
            pltpu.VMEM((NR_W,), jnp.int32),
            pltpu.VMEM((16, SEGW), jnp.float32),
            pltpu.VMEM((SEGW,), jnp.float32),
        ],
    )
    def k(y_hbm, seg_hbm, out_hbm, y_v, seg_v, acc_v, pout_v):
        w = _wid()
        pltpu.sync_copy(y_hbm.at[pl.ds(w * NR_W, NR_W)], y_v)
        pltpu.sync_copy(seg_hbm.at[pl.ds(w * NR_W, NR_W)], seg_v)

        zero16 = jnp.zeros((16,), jnp.float32)

        def zacc(kk, carry):
            for r in range(16):
                acc_v[r, pl.ds(kk * 16, 16)] = zero16
            return carry

        lax.fori_loop(0, SEGW // 16, zacc, 0)

        lane = lax.iota(jnp.int32, 16)
        gbase = lane * PER_LANE

        def step(j, carry):
            cur, acc = carry
            g = gbase + j
            yv = plsc.load_gather(y_v, [g])
            sg = plsc.load_gather(seg_v, [g])
            changed = sg != cur
            plsc.addupdate_scatter(acc_v, [lane, cur], acc, mask=changed)
            acc = jnp.where(changed, yv, acc + yv)
            return sg, acc

        cur0 = jnp.full((16,), 460, jnp.int32)
        cur, acc = lax.fori_loop(0, PER_LANE, step, (cur0, zero16))
        plsc.addupdate_scatter(acc_v, [lane, cur], acc)

        def red(kk, carry):
            t = acc_v[0, pl.ds(kk * 16, 16)]
            for r in range(1, 16):
                t = t + acc_v[r, pl.ds(kk * 16, 16)]
            pout_v[pl.ds(kk * 16, 16)] = t
            return carry

        lax.fori_loop(0, SEGW // 16, red, 0)
        pltpu.sync_copy(pout_v, out_hbm.at[pl.ds(w * SEGW, SEGW)])

    return k(y_f, seg_f)


def _block_diag(w, copies):
    ki, ko = w.shape
    out = jnp.zeros((copies * ki, copies * ko), w.dtype)
    for i in range(copies):
        out = out.at[i * ki:(i + 1) * ki, i * ko:(i + 1) * ko].set(w)
    return out


def _edge_mlp(gs4, gd4, W1b, b1b):
    BLK = 2048
    R = EPAD // 4
    grid = (R // BLK,)

    def body(xs_ref, xd_ref, w_ref, b_ref, o_ref):
        x = xs_ref[...] * xd_ref[...]
        y = jnp.dot(x, w_ref[...], preferred_element_type=jnp.float32)
        o_ref[...] = jnp.maximum(y + b_ref[...], 0.0)

    return pl.pallas_call(
        body,
        grid=grid,
        in_specs=[
            pl.BlockSpec((BLK, 128), lambda i: (i, 0)),
            pl.BlockSpec((BLK, 128), lambda i: (i, 0)),
            pl.BlockSpec((128, 128), lambda i: (0, 0)),
            pl.BlockSpec((1, 128), lambda i: (0, 0)),
        ],
        out_specs=pl.BlockSpec((BLK, 128), lambda i: (i, 0)),
        out_shape=jax.ShapeDtypeStruct((R, 128), jnp.float32),
    )(gs4, gd4, W1b, b1b)


def _node_update(h4, S4, deg4, W2b, b2b, Wu1b, bu1b, Wu2b, bu2b):
    BLK = 1024
    R = NPAD // 4
    grid = (R // BLK,)

    def body(h_ref, s_ref, d_ref, w2_ref, b2_ref, wu1_ref, bu1_ref,
             wu2_ref, bu2_ref, o_ref):
        nm = jnp.dot(s_ref[...], w2_ref[...],
                     preferred_element_type=jnp.float32)
        nm = nm + d_ref[...] * b2_ref[...]
        t = jnp.maximum(
            jnp.dot(nm, wu1_ref[...], preferred_element_type=jnp.float32)
            + bu1_ref[...], 0.0)
        o_ref[...] = (h_ref[...]
                      + jnp.dot(t, wu2_ref[...],
                                preferred_element_type=jnp.float32)
                      + bu2_ref[...])

    wspec = pl.BlockSpec((128, 128), lambda i: (0, 0))
    bspec = pl.BlockSpec((1, 128), lambda i: (0, 0))
    return pl.pallas_call(
        body,
        grid=grid,
        in_specs=[
            pl.BlockSpec((BLK, 128), lambda i: (i, 0)),
            pl.BlockSpec((BLK, 128), lambda i: (i, 0)),
            pl.BlockSpec((BLK, 128), lambda i: (i, 0)),
            wspec, bspec, wspec, bspec, wspec, bspec,
        ],
        out_specs=pl.BlockSpec((BLK, 128), lambda i: (i, 0)),
        out_shape=jax.ShapeDtypeStruct((R, 128), jnp.float32),
    )(h4, S4, deg4, W2b, b2b, Wu1b, bu1b, Wu2b, bu2b)


def _readout(h4, oth4, A, Bm, br1q, C, br2q):
    BLK = 1024
    R = NPAD // 4
    grid = (R // BLK,)

    def body(h_ref, o_ref, a_ref, b_ref, br1_ref, c_ref, br2_ref, out_ref):
        y = (jnp.dot(h_ref[...], a_ref[...], preferred_element_type=jnp.float32)
             + jnp.dot(o_ref[...], b_ref[...], preferred_element_type=jnp.float32)
             + br1_ref[...])
        y = jnp.maximum(y, 0.0)
        out_ref[...] = (jnp.dot(y, c_ref[...], preferred_element_type=jnp.float32)
                        + br2_ref[...])

    return pl.pallas_call(
        body,
        grid=grid,
        in_specs=[
            pl.BlockSpec((BLK, 128), lambda i: (i, 0)),
            pl.BlockSpec((BLK, 64), lambda i: (i, 0)),
            pl.BlockSpec((128, 192), lambda i: (0, 0)),
            pl.BlockSpec((64, 192), lambda i: (0, 0)),
            pl.BlockSpec((1, 192), lambda i: (0, 0)),
            pl.BlockSpec((192, 4), lambda i: (0, 0)),
            pl.BlockSpec((1, 4), lambda i: (0, 0)),
        ],
        out_specs=pl.BlockSpec((BLK, 4), lambda i: (i, 0)),
        out_shape=jax.ShapeDtypeStruct((R, 4), jnp.float32),
    )(h4, oth4, A, Bm, br1q, C, br2q)


def kernel(encoded_atoms, edges, natoms, other_features, embed,
           Wm1, bm1, Wm2, bm2, Wu1, bu1, Wu2, bu2, Wr1, br1, Wr2, br2):
    f32 = jnp.float32
    atoms_f = jnp.pad(encoded_atoms.astype(jnp.int32), (0, NPAD - N))
    src_f = jnp.pad(edges[0].astype(jnp.int32), (0, EPAD - E),
                    constant_values=N)
    dst_f = jnp.pad(edges[1].astype(jnp.int32), (0, EPAD - E),
                    constant_values=N)

    nb = natoms.shape[0]
    seg = jnp.repeat(jnp.arange(nb, dtype=jnp.int32), natoms,
                     total_repeat_length=N)
    seg_f = jnp.pad(seg, (0, NPAD - N), constant_values=450)

    oth = jnp.pad(other_features.astype(f32), ((0, NPAD - N), (0, 0)))
    oth4 = oth.reshape(NPAD // 4, 64)

    W1b = _block_diag(Wm1.astype(f32), 4)
    b1b = jnp.tile(bm1.astype(f32), 4).reshape(1, 128)
    W2b = _block_diag(Wm2.astype(f32), 4)
    b2b = jnp.tile(bm2.astype(f32), 4).reshape(1, 128)
    Wu1b = _block_diag(Wu1.astype(f32), 4)
    bu1b = jnp.tile(bu1.astype(f32), 4).reshape(1, 128)
    Wu2b = _block_diag(Wu2.astype(f32), 4)
    bu2b = jnp.tile(bu2.astype(f32), 4).reshape(1, 128)

    A = _block_diag(Wr1[:D].astype(f32), 4)
    Bm = _block_diag(Wr1[D:].astype(f32), 4)
    br1q = jnp.tile(br1.astype(f32), 4).reshape(1, 192)
    C = _block_diag(Wr2.astype(f32), 4)
    br2q = jnp.tile(br2.astype(f32), 4).reshape(1, 4)

    h = _emb_gather(atoms_f, embed.astype(f32))
    deg4 = _degree(src_f).reshape(NPAD // 4, 128)

    for _ in range(3):
        gs, gd = _gather2(h, src_f, dst_f)
        act = _edge_mlp(gs.reshape(EPAD // 4, 128),
                        gd.reshape(EPAD // 4, 128), W1b, b1b)
        S = _scatter_add(act.reshape(EPAD, D), src_f)
        h4 = _node_update(h.reshape(NPAD // 4, 128),
                          S.reshape(NPAD // 4, 128), deg4,
                          W2b, b2b, Wu1b, bu1b, Wu2b, bu2b)
        h = h4.reshape(NPAD, D)

    y4 = _readout(h.reshape(NPAD // 4, 128), oth4, A, Bm, br1q, C, br2q)
    partials = _segment_sum(y4.reshape(NPAD), seg_f)
    return partials.reshape(NW, SEGW).sum(axis=0)[:nb]


# speedup vs baseline: 1.5822x; 1.0013x over previous
import functools

import jax
import jax.numpy as jnp
from jax import lax
from jax.experimental import pallas as pl
from jax.experimental.pallas import tpu as pltpu
from jax.experimental.pallas import tpu_sc as plsc

N = 100128
E = 1602048
D = 32
NW = 32
NPAD = 102400
NR_W = NPAD // NW
EPAD = 1605632
ER_W = EPAD // NW
ER_T = EPAD // 16
HALF = NPAD // 2
QR = NPAD // 4
SEGW = 464

_mesh_cache = []


def _MESH():
    if not _mesh_cache:
        _mesh_cache.append(plsc.VectorSubcoreMesh(
            core_axis_name="c", subcore_axis_name="s",
            num_cores=2, num_subcores=16))
    return _mesh_cache[0]


def _wid():
    return lax.axis_index("s") * 2 + lax.axis_index("c")


_SC_PARAMS = dict(use_tc_tiling_on_sc=False)


def _emb_gather(atoms_f, embed):
    @functools.partial(
        pl.kernel,
        out_type=jax.ShapeDtypeStruct((NPAD, D), jnp.float32),
        mesh=_MESH(),
        compiler_params=pltpu.CompilerParams(**_SC_PARAMS),
        scratch_types=[
            pltpu.VMEM((NR_W,), jnp.int32),
            pltpu.VMEM((NR_W, D), jnp.float32),
            pltpu.SemaphoreType.DMA,
        ],
    )
    def k(atoms_hbm, embed_hbm, out_hbm, idx_v, rows_v, sem):
        w = _wid()
        pltpu.sync_copy(atoms_hbm.at[pl.ds(w * NR_W, NR_W)], idx_v)
        pltpu.async_copy(embed_hbm.at[idx_v], rows_v, sem).wait()
        pltpu.sync_copy(rows_v, out_hbm.at[pl.ds(w * NR_W, NR_W)])

    return k(atoms_f, embed)


def _gather2(h, src_f, dst_f):
    CR = 896
    NCH = ER_W // CR

    @functools.partial(
        pl.kernel,
        out_type=[jax.ShapeDtypeStruct((EPAD, D), jnp.float32),
                  jax.ShapeDtypeStruct((EPAD, D), jnp.float32)],
        mesh=_MESH(),
        compiler_params=pltpu.CompilerParams(**_SC_PARAMS),
        scratch_types=[
            pltpu.VMEM((CR,), jnp.int32), pltpu.VMEM((CR,), jnp.int32),
            pltpu.VMEM((CR,), jnp.int32), pltpu.VMEM((CR,), jnp.int32),
            pltpu.VMEM((CR, D), jnp.float32), pltpu.VMEM((CR, D), jnp.float32),
            pltpu.VMEM((CR, D), jnp.float32), pltpu.VMEM((CR, D), jnp.float32),
            pltpu.SemaphoreType.DMA,
            pltpu.SemaphoreType.DMA,
        ],
    )
    def k(h_hbm, src_hbm, dst_hbm, gs_hbm, gd_hbm,
          ixs0, ixs1, ixd0, ixd1, rs0, rs1, rd0, rd1, gsem, wsem):
        w = _wid()
        bufs = [(ixs0, ixd0, rs0, rd0), (ixs1, ixd1, rs1, rd1)]

        def chunk(ci, p, drain):
            ixs, ixd, rs, rd = bufs[p]
            e0 = w * ER_W + ci * CR
            pltpu.sync_copy(src_hbm.at[pl.ds(e0, CR)], ixs)
            pltpu.sync_copy(dst_hbm.at[pl.ds(e0, CR)], ixd)
            if drain:
                pltpu.make_async_copy(gs_hbm.at[pl.ds(0, CR)], rs, wsem).wait()
                pltpu.make_async_copy(gs_hbm.at[pl.ds(0, CR)], rd, wsem).wait()
            g1 = pltpu.async_copy(h_hbm.at[ixs], rs, gsem)
            g2 = pltpu.async_copy(h_hbm.at[ixd], rd, gsem)
            g1.wait()
            g2.wait()
            pltpu.async_copy(rs, gs_hbm.at[pl.ds(e0, CR)], wsem)
            pltpu.async_copy(rd, gd_hbm.at[pl.ds(e0, CR)], wsem)

        chunk(0, 0, False)
        chunk(1, 1, False)

        @pl.loop(2, NCH, step=2)
        def _loop(base):
            chunk(base, 0, True)
            chunk(base + 1, 1, True)

        for p in range(2):
            _, _, rs, rd = bufs[p]
            pltpu.make_async_copy(gs_hbm.at[pl.ds(0, CR)], rs, wsem).wait()
            pltpu.make_async_copy(gs_hbm.at[pl.ds(0, CR)], rd, wsem).wait()

    return k(h, src_f, dst_f)


def _scatter_part(act, src_f, kbase):
    CR = 2048
    NCH = ER_T // CR
    ACC = QR + 256
    TR = ACC // 16
    WB = 800
    CPAD = CR + 160

    @functools.partial(
        pl.kernel,
        out_type=jax.ShapeDtypeStruct((2 * QR, D), jnp.float32),
        mesh=_MESH(),
        compiler_params=pltpu.CompilerParams(
            needs_layout_passes=False, **_SC_PARAMS),
        scratch_types=[
            pltpu.VMEM((CR,), jnp.int32),
            pltpu.VMEM((CPAD,), jnp.int32),
            pltpu.VMEM((CPAD,), jnp.int32),
            pltpu.VMEM((TR, D), jnp.float32),
            pltpu.VMEM_SHARED((ACC, D), jnp.float32),
            pltpu.SemaphoreType.DMA,
            pltpu.SemaphoreType.DMA,
        ],
    )
    def k(act_hbm, src_hbm, out_hbm, ix, ce, cl, vv, acc_sh, asem, gsem):
        c = lax.axis_index("c")
        s = lax.axis_index("s")
        base = kbase + c * QR
        zero16 = jnp.zeros((16,), jnp.float32)
        zero16i = jnp.zeros((16,), jnp.int32)
        dump16 = jnp.full((16,), QR, jnp.int32)
        iota16 = lax.iota(jnp.int32, 16)

        def zrow(r, carry):
            vv[r, pl.ds(0, 16)] = zero16
            vv[r, pl.ds(16, 16)] = zero16
            return carry

        lax.fori_loop(0, TR, zrow, 0)
        pltpu.sync_copy(vv, acc_sh.at[pl.ds(s * TR, TR)])
        plsc.subcore_barrier()

        def chunk(ci, carry):
            e0 = (s * NCH + ci) * CR
            pltpu.sync_copy(src_hbm.at[pl.ds(e0, CR)], ix)

            def sc16(t, cnt):
                v = ix[pl.ds(t * 16, 16)]
                ok = (v >= base) & (v < base + QR)
                plsc.store_compressed(ce.at[pl.ds(cnt, 16)],
                                      iota16 + (e0 + t * 16), mask=ok)
                plsc.store_compressed(cl.at[pl.ds(cnt, 16)], v - base,
                                      mask=ok)
                return cnt + jnp.sum(ok.astype(jnp.int32))

            cnt = lax.fori_loop(0, CR // 16, sc16, 0)
            for j in range(8):
                ce[pl.ds(cnt + j * 16, 16)] = zero16i
                cl[pl.ds(cnt + j * 16, 16)] = dump16
            for j in range(16):
                off = 256 + (j % 2) * 128

                @pl.when(j * 128 < cnt)
                def _():
                    if j >= 2:
                        pltpu.make_async_copy(
                            act_hbm.at[pl.ds(0, 128)],
                            vv.at[pl.ds(off, 128)], asem).wait()
                    pltpu.async_copy(
                        act_hbm.at[ce.at[pl.ds(j * 128, 128)]],
                        vv.at[pl.ds(off, 128)], gsem).wait()
                    pltpu.async_copy(
                        vv.at[pl.ds(off, 128)],
                        acc_sh.at[cl.at[pl.ds(j * 128, 128)]],
                        asem, add=True)
            for j in range(16):
                @pl.when((j * 128 < cnt) & ((j + 2) * 128 >= cnt))
                def _():
                    pltpu.make_async_copy(
                        act_hbm.at[pl.ds(0, 128)],
                        vv.at[pl.ds(256 + (j % 2) * 128, 128)], asem).wait()
            return carry

        lax.fori_loop(0, NCH, chunk, 0)
        plsc.subcore_barrier()

        for t in range(2):
            pltpu.sync_copy(acc_sh.at[pl.ds(s * (2 * WB) + t * WB, WB)],
                            vv.at[pl.ds(0, WB)])
            pltpu.sync_copy(vv.at[pl.ds(0, WB)],
                            out_hbm.at[pl.ds(c * QR + s * (2 * WB) + t * WB, WB)])

    return k(act, src_f)


def _scatter_add(act, src_f):
    lo = _scatter_part(act, src_f, 0)
    hi = _scatter_part(act, src_f, HALF)
    return jnp.concatenate([lo, hi], axis=0)


def _degree_part(src_f, kbase):
    CR = 2048
    NCH = ER_T // CR
    ACC = QR + 256
    TR = ACC // 16
    WB = 800
    CPAD = CR + 160

    @functools.partial(
        pl.kernel,
        out_type=jax.ShapeDtypeStruct((2 * QR, D), jnp.float32),
        mesh=_MESH(),
        compiler_params=pltpu.CompilerParams(
            needs_layout_passes=False, **_SC_PARAMS),
        scratch_types=[
            pltpu.VMEM((CR,), jnp.int32),
            pltpu.VMEM((CPAD,), jnp.int32),
            pltpu.VMEM((TR, D), jnp.float32),
            pltpu.VMEM_SHARED((ACC, D), jnp.float32),
            pltpu.SemaphoreType.DMA,
        ],
    )
    def k(src_hbm, out_hbm, ix, cl, ones_v, acc_sh, asem):
        c = lax.axis_index("c")
        s = lax.axis_index("s")
        base = kbase + c * QR
        zero16 = jnp.zeros((16,), jnp.float32)
        one16 = jnp.ones((16,), jnp.float32)
        dump16 = jnp.full((16,), QR, jnp.int32)

        def zrow(r, carry):
            ones_v[r, pl.ds(0, 16)] = zero16
            ones_v[r, pl.ds(16, 16)] = zero16
            return carry

        lax.fori_loop(0, TR, zrow, 0)
        pltpu.sync_copy(ones_v, acc_sh.at[pl.ds(s * TR, TR)])

        def orow(r, carry):
            ones_v[r, pl.ds(0, 16)] = one16
            ones_v[r, pl.ds(16, 16)] = one16
            return carry

        lax.fori_loop(0, 128, orow, 0)
        plsc.subcore_barrier()

        def chunk(ci, carry):
            e0 = (s * NCH + ci) * CR
            pltpu.sync_copy(src_hbm.at[pl.ds(e0, CR)], ix)

            def sc16(t, cnt):
                v = ix[pl.ds(t * 16, 16)]
                ok = (v >= base) & (v < base + QR)
                plsc.store_compressed(cl.at[pl.ds(cnt, 16)], v - base,
                                      mask=ok)
                return cnt + jnp.sum(ok.astype(jnp.int32))

            cnt = lax.fori_loop(0, CR // 16, sc16, 0)
            for j in range(8):
                cl[pl.ds(cnt + j * 16, 16)] = dump16
            for j in range(16):
                @pl.when(j * 128 < cnt)
                def _():
                    pltpu.async_copy(
                        ones_v.at[pl.ds(0, 128)],
                        acc_sh.at[cl.at[pl.ds(j * 128, 128)]],
                        asem, add=True).wait()
            return carry

        lax.fori_loop(0, NCH, chunk, 0)
        plsc.subcore_barrier()

        for t in range(2):
            pltpu.sync_copy(acc_sh.at[pl.ds(s * (2 * WB) + t * WB, WB)],
                            ones_v.at[pl.ds(0, WB)])
            pltpu.sync_copy(ones_v.at[pl.ds(0, WB)],
                            out_hbm.at[pl.ds(c * QR + s * (2 * WB) + t * WB, WB)])

    return k(src_f)


def _degree(src_f):
    lo = _degree_part(src_f, 0)
    hi = _degree_part(src_f, HALF)
    return jnp.concatenate([lo, hi], axis=0)


def _segment_sum(y_f, seg_f):
    PER_LANE = NR_W // 16

    @functools.partial(
        pl.kernel,
        out_type=jax.ShapeDtypeStruct((NW * SEGW,), jnp.float32),
        mesh=_MESH(),
        compiler_params=pltpu.CompilerParams(
            needs_layout_passes=False, **_SC_PARAMS),
        scratch_types=[
            pltpu.VMEM((NR_W,), jnp.float32),
            pltpu.VMEM((NR_W,), jnp.int32),
            pltpu.VMEM((16, SEGW), jnp.float32),
            pltpu.VMEM((SEGW,), jnp.float32),
        ],
    )
    def k(y_hbm, seg_hbm, out_hbm, y_v, seg_v, acc_v, pout_v):
        w = _wid()
        pltpu.sync_copy(y_hbm.at[pl.ds(w * NR_W, NR_W)], y_v)
        pltpu.sync_copy(seg_hbm.at[pl.ds(w * NR_W, NR_W)], seg_v)

        zero16 = jnp.zeros((16,), jnp.float32)

        def zacc(kk, carry):
            for r in range(16):
                acc_v[r, pl.ds(kk * 16, 16)] = zero16
            return carry

        lax.fori_loop(0, SEGW // 16, zacc, 0)

        lane = lax.iota(jnp.int32, 16)
        gbase = lane * PER_LANE

        def step(j, carry):
            cur, acc = carry
            g = gbase + j
            yv = plsc.load_gather(y_v, [g])
            sg = plsc.load_gather(seg_v, [g])
            changed = sg != cur
            plsc.addupdate_scatter(acc_v, [lane, cur], acc, mask=changed)
            acc = jnp.where(changed, yv, acc + yv)
            return sg, acc

        cur0 = jnp.full((16,), 460, jnp.int32)
        cur, acc = lax.fori_loop(0, PER_LANE, step, (cur0, zero16))
        plsc.addupdate_scatter(acc_v, [lane, cur], acc)

        def red(kk, carry):
            t = acc_v[0, pl.ds(kk * 16, 16)]
            for r in range(1, 16):
                t = t + acc_v[r, pl.ds(kk * 16, 16)]
            pout_v[pl.ds(kk * 16, 16)] = t
            return carry

        lax.fori_loop(0, SEGW // 16, red, 0)
        pltpu.sync_copy(pout_v, out_hbm.at[pl.ds(w * SEGW, SEGW)])

    return k(y_f, seg_f)


def _block_diag(w, copies):
    ki, ko = w.shape
    out = jnp.zeros((copies * ki, copies * ko), w.dtype)
    for i in range(copies):
        out = out.at[i * ki:(i + 1) * ki, i * ko:(i + 1) * ko].set(w)
    return out


def _edge_mlp(gs4, gd4, W1b, b1b):
    BLK = 2048
    R = EPAD // 4
    grid = (R // BLK,)

    def body(xs_ref, xd_ref, w_ref, b_ref, o_ref):
        x = xs_ref[...] * xd_ref[...]
        y = jnp.dot(x, w_ref[...], preferred_element_type=jnp.float32)
        o_ref[...] = jnp.maximum(y + b_ref[...], 0.0)

    return pl.pallas_call(
        body,
        grid=grid,
        in_specs=[
            pl.BlockSpec((BLK, 128), lambda i: (i, 0)),
            pl.BlockSpec((BLK, 128), lambda i: (i, 0)),
            pl.BlockSpec((128, 128), lambda i: (0, 0)),
            pl.BlockSpec((1, 128), lambda i: (0, 0)),
        ],
        out_specs=pl.BlockSpec((BLK, 128), lambda i: (i, 0)),
        out_shape=jax.ShapeDtypeStruct((R, 128), jnp.float32),
    )(gs4, gd4, W1b, b1b)


def _node_update(h4, S4, deg4, W2b, b2b, Wu1b, bu1b, Wu2b, bu2b):
    BLK = 1024
    R = NPAD // 4
    grid = (R // BLK,)

    def body(h_ref, s_ref, d_ref, w2_ref, b2_ref, wu1_ref, bu1_ref,
             wu2_ref, bu2_ref, o_ref):
        nm = jnp.dot(s_ref[...], w2_ref[...],
                     preferred_element_type=jnp.float32)
        nm = nm + d_ref[...] * b2_ref[...]
        t = jnp.maximum(
            jnp.dot(nm, wu1_ref[...], preferred_element_type=jnp.float32)
            + bu1_ref[...], 0.0)
        o_ref[...] = (h_ref[...]
                      + jnp.dot(t, wu2_ref[...],
                                preferred_element_type=jnp.float32)
                      + bu2_ref[...])

    wspec = pl.BlockSpec((128, 128), lambda i: (0, 0))
    bspec = pl.BlockSpec((1, 128), lambda i: (0, 0))
    return pl.pallas_call(
        body,
        grid=grid,
        in_specs=[
            pl.BlockSpec((BLK, 128), lambda i: (i, 0)),
            pl.BlockSpec((BLK, 128), lambda i: (i, 0)),
            pl.BlockSpec((BLK, 128), lambda i: (i, 0)),
            wspec, bspec, wspec, bspec, wspec, bspec,
        ],
        out_specs=pl.BlockSpec((BLK, 128), lambda i: (i, 0)),
        out_shape=jax.ShapeDtypeStruct((R, 128), jnp.float32),
    )(h4, S4, deg4, W2b, b2b, Wu1b, bu1b, Wu2b, bu2b)


def _readout(h4, oth4, A, Bm, br1q, C, br2q):
    BLK = 1024
    R = NPAD // 4
    grid = (R // BLK,)

    def body(h_ref, o_ref, a_ref, b_ref, br1_ref, c_ref, br2_ref, out_ref):
        y = (jnp.dot(h_ref[...], a_ref[...], preferred_element_type=jnp.float32)
             + jnp.dot(o_ref[...], b_ref[...], preferred_element_type=jnp.float32)
             + br1_ref[...])
        y = jnp.maximum(y, 0.0)
        out_ref[...] = (jnp.dot(y, c_ref[...], preferred_element_type=jnp.float32)
                        + br2_ref[...])

    return pl.pallas_call(
        body,
        grid=grid,
        in_specs=[
            pl.BlockSpec((BLK, 128), lambda i: (i, 0)),
            pl.BlockSpec((BLK, 64), lambda i: (i, 0)),
            pl.BlockSpec((128, 192), lambda i: (0, 0)),
            pl.BlockSpec((64, 192), lambda i: (0, 0)),
            pl.BlockSpec((1, 192), lambda i: (0, 0)),
            pl.BlockSpec((192, 4), lambda i: (0, 0)),
            pl.BlockSpec((1, 4), lambda i: (0, 0)),
        ],
        out_specs=pl.BlockSpec((BLK, 4), lambda i: (i, 0)),
        out_shape=jax.ShapeDtypeStruct((R, 4), jnp.float32),
    )(h4, oth4, A, Bm, br1q, C, br2q)


def kernel(encoded_atoms, edges, natoms, other_features, embed,
           Wm1, bm1, Wm2, bm2, Wu1, bu1, Wu2, bu2, Wr1, br1, Wr2, br2):
    f32 = jnp.float32
    atoms_f = jnp.pad(encoded_atoms.astype(jnp.int32), (0, NPAD - N))
    src_f = jnp.pad(edges[0].astype(jnp.int32), (0, EPAD - E),
                    constant_values=N)
    dst_f = jnp.pad(edges[1].astype(jnp.int32), (0, EPAD - E),
                    constant_values=N)

    nb = natoms.shape[0]
    seg = jnp.repeat(jnp.arange(nb, dtype=jnp.int32), natoms,
                     total_repeat_length=N)
    seg_f = jnp.pad(seg, (0, NPAD - N), constant_values=450)

    oth = jnp.pad(other_features.astype(f32), ((0, NPAD - N), (0, 0)))
    oth4 = oth.reshape(NPAD // 4, 64)

    W1b = _block_diag(Wm1.astype(f32), 4)
    b1b = jnp.tile(bm1.astype(f32), 4).reshape(1, 128)
    W2b = _block_diag(Wm2.astype(f32), 4)
    b2b = jnp.tile(bm2.astype(f32), 4).reshape(1, 128)
    Wu1b = _block_diag(Wu1.astype(f32), 4)
    bu1b = jnp.tile(bu1.astype(f32), 4).reshape(1, 128)
    Wu2b = _block_diag(Wu2.astype(f32), 4)
    bu2b = jnp.tile(bu2.astype(f32), 4).reshape(1, 128)

    A = _block_diag(Wr1[:D].astype(f32), 4)
    Bm = _block_diag(Wr1[D:].astype(f32), 4)
    br1q = jnp.tile(br1.astype(f32), 4).reshape(1, 192)
    C = _block_diag(Wr2.astype(f32), 4)
    br2q = jnp.tile(br2.astype(f32), 4).reshape(1, 4)

    h = _emb_gather(atoms_f, embed.astype(f32))
    deg4 = _degree(src_f).reshape(NPAD // 4, 128)

    for _ in range(3):
        gs, gd = _gather2(h, src_f, dst_f)
        act = _edge_mlp(gs.reshape(EPAD // 4, 128),
                        gd.reshape(EPAD // 4, 128), W1b, b1b)
        S = _scatter_add(act.reshape(EPAD, D), src_f)
        h4 = _node_update(h.reshape(NPAD // 4, 128),
                          S.reshape(NPAD // 4, 128), deg4,
                          W2b, b2b, Wu1b, bu1b, Wu2b, bu2b)
        h = h4.reshape(NPAD, D)

    y4 = _readout(h.reshape(NPAD // 4, 128), oth4, A, Bm, br1q, C, br2q)
    partials = _segment_sum(y4.reshape(NPAD), seg_f)
    return partials.reshape(NW, SEGW).sum(axis=0)[:nb]
